# baseline (speedup 1.0000x reference)
import functools

import jax
import jax.numpy as jnp
from jax import lax
from jax.experimental import pallas as pl
from jax.experimental.pallas import tpu as pltpu
from jax.experimental.pallas import tpu_sc as plsc

N = 10000
E = 320000
H = 128
EF = 16

NC = 2
NS = 16
NW = NC * NS
EPW = E // NW
CH = 80
NCHUNK = EPW // CH
RCP = 80
NRC = N // RCP
F32 = jnp.float32


def _prep_body(x_ref, c_ref, ws_ref, wd_ref, wc_ref, b1_ref, a_ref, b_ref):
    x = x_ref[...]
    cw = jnp.dot(c_ref[...], wc_ref[...], preferred_element_type=F32)
    a_ref[...] = jnp.dot(x, ws_ref[...], preferred_element_type=F32) - cw
    b_ref[...] = jnp.dot(x, wd_ref[...], preferred_element_type=F32) + cw + b1_ref[...]


def _prep(x, coords_p, ws, wd, wc_p, b1):
    blk = 1000
    grid = N // blk
    return pl.pallas_call(
        _prep_body,
        grid=(grid,),
        in_specs=[
            pl.BlockSpec((blk, H), lambda i: (i, 0)),
            pl.BlockSpec((blk, 8), lambda i: (i, 0)),
            pl.BlockSpec((H, H), lambda i: (0, 0)),
            pl.BlockSpec((H, H), lambda i: (0, 0)),
            pl.BlockSpec((8, H), lambda i: (0, 0)),
            pl.BlockSpec((1, H), lambda i: (0, 0)),
        ],
        out_specs=[
            pl.BlockSpec((blk, H), lambda i: (i, 0)),
            pl.BlockSpec((blk, H), lambda i: (i, 0)),
        ],
        out_shape=[
            jax.ShapeDtypeStruct((N, H), F32),
            jax.ShapeDtypeStruct((N, H), F32),
        ],
    )(x, coords_p, ws, wd, wc_p, b1)


CHG = 40
RPE = 2 * CHG
NCG = EPW // CHG
NSLG = 5


def _gather_body(t_hbm, idx_hbm, g_hbm, *s):
    idxv = s[0]
    rows = s[1:1 + NSLG]
    gv = s[1 + NSLG:1 + 2 * NSLG]
    sg = s[1 + 2 * NSLG:1 + 3 * NSLG]
    sw = s[1 + 3 * NSLG:1 + 4 * NSLG]
    wid = lax.axis_index("s") * NC + lax.axis_index("c")
    base = wid * EPW
    ibase = pl.multiple_of(wid * 2 * EPW, 8)

    def off_of(t):
        return pl.multiple_of(base + t * CHG, 8)

    def islice(t):
        return idxv.at[pl.ds(pl.multiple_of(t * RPE, 8), RPE)]

    pltpu.sync_copy(idx_hbm.at[pl.ds(ibase, 2 * EPW)], idxv)
    for b in range(NSLG):
        pltpu.make_async_copy(t_hbm.at[islice(b)], rows[b], sg[b]).start()

    def outer(q, carry):
        for b in range(NSLG):
            t = NSLG * q + b
            pltpu.make_async_copy(t_hbm.at[islice(t)], rows[b], sg[b]).wait()
            r_b, gv_b = rows[b], gv[b]

            @pl.when(q > 0)
            def _():
                pltpu.make_async_copy(gv_b, g_hbm.at[pl.ds(off_of(t - NSLG), CHG)],
                                      sw[b]).wait()

            def row(e, c2):
                for k in range(8):
                    sl = pl.ds(k * 16, 16)
                    gv_b[e, sl] = r_b[2 * e, sl] + r_b[2 * e + 1, sl]
                return c2

            lax.fori_loop(0, CHG, row, 0, unroll=2)
            pltpu.make_async_copy(gv_b, g_hbm.at[pl.ds(off_of(t), CHG)], sw[b]).start()

            @pl.when(t + NSLG < NCG)
            def _():
                pltpu.make_async_copy(t_hbm.at[islice(t + NSLG)], rows[b], sg[b]).start()
        return carry

    lax.fori_loop(0, NCG // NSLG, outer, 0)
    for b in range(NSLG):
        pltpu.make_async_copy(gv[b], g_hbm.at[pl.ds(off_of(NCG - NSLG + b), CHG)],
                              sw[b]).wait()


_gather = functools.partial(
    pl.kernel,
    out_type=jax.ShapeDtypeStruct((E, H), F32),
    mesh=plsc.VectorSubcoreMesh(core_axis_name="c", subcore_axis_name="s",
                                num_cores=NC, num_subcores=NS),
    scratch_types=(
        [pltpu.VMEM((2 * EPW,), jnp.int32)]
        + [pltpu.VMEM((RPE, H), F32) for _ in range(NSLG)]
        + [pltpu.VMEM((CHG, H), F32) for _ in range(NSLG)]
        + [pltpu.SemaphoreType.DMA for _ in range(2 * NSLG)]
    ),
)(_gather_body)


def _edge_body(g_ref, ea_ref, we_ref, w2_ref, b2_ref, m_ref):
    h1 = jnp.maximum(
        g_ref[...] + jnp.dot(ea_ref[...], we_ref[...], preferred_element_type=F32),
        0.0)
    m_ref[...] = jnp.maximum(
        jnp.dot(h1, w2_ref[...], preferred_element_type=F32) + b2_ref[...],
        0.0)


def _edge(g, ea, we, w2, b2):
    blk = 3200
    grid = E // blk
    return pl.pallas_call(
        _edge_body,
        grid=(grid,),
        in_specs=[
            pl.BlockSpec((blk, H), lambda i: (i, 0)),
            pl.BlockSpec((blk, EF), lambda i: (i, 0)),
            pl.BlockSpec((EF, H), lambda i: (0, 0)),
            pl.BlockSpec((H, H), lambda i: (0, 0)),
            pl.BlockSpec((1, H), lambda i: (0, 0)),
        ],
        out_specs=pl.BlockSpec((blk, H), lambda i: (i, 0)),
        out_shape=jax.ShapeDtypeStruct((E, H), F32),
    )(g, ea, we, w2, b2)


CHS = 40
NCS = EPW // CHS
NSL = 5


def _scatter_body(msg_hbm, dst_hbm, out_hbm, *s):
    idxs = s[0:NSL]
    mv = s[NSL:2 * NSL]
    zv = s[2 * NSL]
    accum = s[2 * NSL + 1]
    sli = s[2 * NSL + 2:2 * NSL + 2 + NSL]
    slm = s[2 * NSL + 2 + NSL:2 * NSL + 2 + 2 * NSL]
    sad = s[2 * NSL + 2 + 2 * NSL:2 * NSL + 2 + 3 * NSL]
    cid = lax.axis_index("c")
    sid = lax.axis_index("s")
    base = cid * (E // NC) + sid * EPW

    def off_of(t):
        return pl.multiple_of(base + t * CHS, 8)

    def zrow(r, c2):
        for k in range(8):
            zv[r, pl.ds(k * 16, 16)] = jnp.zeros((16,), F32)
        return c2

    lax.fori_loop(0, RCP, zrow, 0)

    nmine = (NRC - 1 - sid) // NS + 1

    def zcopy(j, c2):
        r0 = pl.multiple_of((sid + j * NS) * RCP, 8)
        pltpu.sync_copy(zv, accum.at[pl.ds(r0, RCP)])
        return c2

    lax.fori_loop(0, nmine, zcopy, 0)
    plsc.subcore_barrier()

    for b in range(NSL):
        off = off_of(b)
        pltpu.make_async_copy(dst_hbm.at[pl.ds(off, CHS)], idxs[b], sli[b]).start()
        pltpu.make_async_copy(msg_hbm.at[pl.ds(off, CHS)], mv[b], slm[b]).start()

    def outer(g, carry):
        for b in range(NSL):
            t = NSL * g + b
            off = off_of(t)
            pltpu.make_async_copy(dst_hbm.at[pl.ds(off, CHS)], idxs[b], sli[b]).wait()
            pltpu.make_async_copy(msg_hbm.at[pl.ds(off, CHS)], mv[b], slm[b]).wait()
            pltpu.async_copy(mv[b], accum.at[idxs[b]], sad[b], add=True)
        for b in range(NSL):
            t = NSL * g + b

            @pl.when(t + NSL < NCS)
            def _():
                noff = off_of(t + NSL)
                pltpu.make_async_copy(mv[b], accum.at[idxs[b]], sad[b]).wait()
                pltpu.make_async_copy(dst_hbm.at[pl.ds(noff, CHS)], idxs[b], sli[b]).start()
                pltpu.make_async_copy(msg_hbm.at[pl.ds(noff, CHS)], mv[b], slm[b]).start()
        return carry

    lax.fori_loop(0, NCS // NSL, outer, 0)
    for b in range(NSL):
        pltpu.make_async_copy(mv[b], accum.at[idxs[b]], sad[b]).wait()
    plsc.subcore_barrier()

    def ocopy(j, c2):
        r0 = pl.multiple_of((sid + j * NS) * RCP, 8)
        pltpu.sync_copy(accum.at[pl.ds(r0, RCP)],
                        out_hbm.at[pl.ds(pl.multiple_of(cid * N + r0, 8), RCP)])
        return c2

    lax.fori_loop(0, nmine, ocopy, 0)


_scatter = functools.partial(
    pl.kernel,
    out_type=jax.ShapeDtypeStruct((2 * N, H), F32),
    mesh=plsc.VectorSubcoreMesh(core_axis_name="c", subcore_axis_name="s",
                                num_cores=NC, num_subcores=NS),
    scratch_types=(
        [pltpu.VMEM((CHS,), jnp.int32) for _ in range(NSL)]
        + [pltpu.VMEM((CHS, H), F32) for _ in range(NSL)]
        + [pltpu.VMEM((RCP, H), F32), pltpu.VMEM_SHARED((N, H), F32)]
        + [pltpu.SemaphoreType.DMA for _ in range(3 * NSL)]
    ),
)(_scatter_body)


def _node_body(x_ref, p_ref, u1a_ref, u1b_ref, b1_ref, u2_ref, b2_ref,
               g_ref, bb_ref, o_ref):
    x = x_ref[...]
    aggr = p_ref[0, :, :] + p_ref[1, :, :]
    h = jnp.maximum(
        jnp.dot(x, u1a_ref[...], preferred_element_type=F32)
        + jnp.dot(aggr, u1b_ref[...], preferred_element_type=F32)
        + b1_ref[...], 0.0)
    o = jnp.maximum(jnp.dot(h, u2_ref[...], preferred_element_type=F32)
                    + b2_ref[...], 0.0)
    y = x + o
    mu = jnp.mean(y, axis=1, keepdims=True)
    var = jnp.mean((y - mu) * (y - mu), axis=1, keepdims=True)
    o_ref[...] = (y - mu) * lax.rsqrt(var + 1e-5) * g_ref[...] + bb_ref[...]


def _node(x, p, u1a, u1b, b1, u2, b2, g, b):
    blk = 1000
    grid = N // blk
    return pl.pallas_call(
        _node_body,
        grid=(grid,),
        in_specs=[
            pl.BlockSpec((blk, H), lambda i: (i, 0)),
            pl.BlockSpec((2, blk, H), lambda i: (0, i, 0)),
            pl.BlockSpec((H, H), lambda i: (0, 0)),
            pl.BlockSpec((H, H), lambda i: (0, 0)),
            pl.BlockSpec((1, H), lambda i: (0, 0)),
            pl.BlockSpec((H, H), lambda i: (0, 0)),
            pl.BlockSpec((1, H), lambda i: (0, 0)),
            pl.BlockSpec((1, H), lambda i: (0, 0)),
            pl.BlockSpec((1, H), lambda i: (0, 0)),
        ],
        out_specs=pl.BlockSpec((blk, H), lambda i: (i, 0)),
        out_shape=jax.ShapeDtypeStruct((N, H), F32),
    )(x, p, u1a, u1b, b1, u2, b2, g, b)


def kernel(x, edge_index, edge_attr, coords,
           msg_W1, msg_b1, msg_W2, msg_b2,
           upd_W1, upd_b1, upd_W2, upd_b2,
           ln_g, ln_b):
    src = edge_index[0]
    dst = edge_index[1]
    ws = msg_W1[:H]
    wd = msg_W1[H:2 * H]
    we = msg_W1[2 * H:2 * H + EF]
    wc = msg_W1[2 * H + EF:]
    coords_p = jnp.pad(coords, ((0, 0), (0, 5)))
    wc_p = jnp.pad(wc, ((0, 5), (0, 0)))

    a, b = _prep(x, coords_p, ws, wd, wc_p, msg_b1.reshape(1, H))
    t = jnp.concatenate([a, b], axis=0)
    idx2 = jnp.stack([src, dst + N], axis=1).reshape(2 * E)
    g = _gather(t, idx2)
    msg = _edge(g, edge_attr, we, msg_W2, msg_b2.reshape(1, H))
    p = _scatter(msg, dst).reshape(2, N, H)
    return _node(x, p, upd_W1[:H], upd_W1[H:], upd_b1.reshape(1, H),
                 upd_W2, upd_b2.reshape(1, H),
                 ln_g.reshape(1, H), ln_b.reshape(1, H))


# speedup vs baseline: 1.2559x; 1.2559x over previous
import functools

import jax
import jax.numpy as jnp
from jax import lax
from jax.experimental import pallas as pl
from jax.experimental.pallas import tpu as pltpu
from jax.experimental.pallas import tpu_sc as plsc

N = 10000
E = 320000
H = 128
EF = 16

NC = 2
NS = 16
NW = NC * NS
EPW = E // NW
CH = 80
NCHUNK = EPW // CH
RCP = 80
NRC = N // RCP
F32 = jnp.float32


def _prep_body(x_ref, c_ref, ws_ref, wd_ref, wc_ref, b1_ref, a_ref, b_ref):
    x = x_ref[...]
    cw = jnp.dot(c_ref[...], wc_ref[...], preferred_element_type=F32)
    a_ref[...] = jnp.dot(x, ws_ref[...], preferred_element_type=F32) - cw
    b_ref[...] = jnp.dot(x, wd_ref[...], preferred_element_type=F32) + cw + b1_ref[...]


def _prep(x, coords_p, ws, wd, wc_p, b1):
    blk = 1000
    grid = N // blk
    return pl.pallas_call(
        _prep_body,
        grid=(grid,),
        in_specs=[
            pl.BlockSpec((blk, H), lambda i: (i, 0)),
            pl.BlockSpec((blk, 8), lambda i: (i, 0)),
            pl.BlockSpec((H, H), lambda i: (0, 0)),
            pl.BlockSpec((H, H), lambda i: (0, 0)),
            pl.BlockSpec((8, H), lambda i: (0, 0)),
            pl.BlockSpec((1, H), lambda i: (0, 0)),
        ],
        out_specs=[
            pl.BlockSpec((blk, H), lambda i: (i, 0)),
            pl.BlockSpec((blk, H), lambda i: (i, 0)),
        ],
        out_shape=[
            jax.ShapeDtypeStruct((N, H), F32),
            jax.ShapeDtypeStruct((N, H), F32),
        ],
    )(x, coords_p, ws, wd, wc_p, b1)


CHG = 40
NCG = EPW // CHG
NSLG = 5


def _gather_body(a_hbm, b_hbm, src_hbm, dst_hbm, g_hbm, *s):
    isrc = s[0]
    idst = s[1]
    ra = s[2:2 + NSLG]
    rb = s[2 + NSLG:2 + 2 * NSLG]
    gv = s[2 + 2 * NSLG:2 + 3 * NSLG]
    sa = s[2 + 3 * NSLG:2 + 4 * NSLG]
    sb = s[2 + 4 * NSLG:2 + 5 * NSLG]
    sw = s[2 + 5 * NSLG:2 + 6 * NSLG]
    wid = lax.axis_index("s") * NC + lax.axis_index("c")
    base = wid * EPW

    def off_of(t):
        return pl.multiple_of(base + t * CHG, 8)

    def sslice(t):
        return isrc.at[pl.ds(pl.multiple_of(t * CHG, 8), CHG)]

    def dslice(t):
        return idst.at[pl.ds(pl.multiple_of(t * CHG, 8), CHG)]

    pltpu.sync_copy(src_hbm.at[pl.ds(pl.multiple_of(base, 8), EPW)], isrc)
    pltpu.sync_copy(dst_hbm.at[pl.ds(pl.multiple_of(base, 8), EPW)], idst)
    for b in range(NSLG):
        pltpu.make_async_copy(a_hbm.at[sslice(b)], ra[b], sa[b]).start()
        pltpu.make_async_copy(b_hbm.at[dslice(b)], rb[b], sb[b]).start()

    def outer(q, carry):
        for b in range(NSLG):
            t = NSLG * q + b
            pltpu.make_async_copy(a_hbm.at[sslice(t)], ra[b], sa[b]).wait()
            pltpu.make_async_copy(b_hbm.at[dslice(t)], rb[b], sb[b]).wait()
            ra_b, rb_b, gv_b = ra[b], rb[b], gv[b]

            @pl.when(q > 0)
            def _():
                pltpu.make_async_copy(gv_b, g_hbm.at[pl.ds(off_of(t - NSLG), CHG)],
                                      sw[b]).wait()

            def row(e, c2):
                for k in range(8):
                    sl = pl.ds(k * 16, 16)
                    gv_b[e, sl] = ra_b[e, sl] + rb_b[e, sl]
                return c2

            lax.fori_loop(0, CHG, row, 0, unroll=2)
            pltpu.make_async_copy(gv_b, g_hbm.at[pl.ds(off_of(t), CHG)], sw[b]).start()

            @pl.when(t + NSLG < NCG)
            def _():
                pltpu.make_async_copy(a_hbm.at[sslice(t + NSLG)], ra[b], sa[b]).start()
                pltpu.make_async_copy(b_hbm.at[dslice(t + NSLG)], rb[b], sb[b]).start()
        return carry

    lax.fori_loop(0, NCG // NSLG, outer, 0)
    for b in range(NSLG):
        pltpu.make_async_copy(gv[b], g_hbm.at[pl.ds(off_of(NCG - NSLG + b), CHG)],
                              sw[b]).wait()


_gather = functools.partial(
    pl.kernel,
    out_type=jax.ShapeDtypeStruct((E, H), F32),
    mesh=plsc.VectorSubcoreMesh(core_axis_name="c", subcore_axis_name="s",
                                num_cores=NC, num_subcores=NS),
    scratch_types=(
        [pltpu.VMEM((EPW,), jnp.int32) for _ in range(2)]
        + [pltpu.VMEM((CHG, H), F32) for _ in range(3 * NSLG)]
        + [pltpu.SemaphoreType.DMA for _ in range(3 * NSLG)]
    ),
)(_gather_body)


def _edge_body(g_ref, ea_ref, we_ref, w2_ref, b2_ref, m_ref):
    h1 = jnp.maximum(
        g_ref[...] + jnp.dot(ea_ref[...], we_ref[...], preferred_element_type=F32),
        0.0)
    m_ref[...] = jnp.maximum(
        jnp.dot(h1, w2_ref[...], preferred_element_type=F32) + b2_ref[...],
        0.0)


def _edge(g, ea, we, w2, b2):
    blk = 3200
    grid = E // blk
    return pl.pallas_call(
        _edge_body,
        grid=(grid,),
        in_specs=[
            pl.BlockSpec((blk, H), lambda i: (i, 0)),
            pl.BlockSpec((blk, EF), lambda i: (i, 0)),
            pl.BlockSpec((EF, H), lambda i: (0, 0)),
            pl.BlockSpec((H, H), lambda i: (0, 0)),
            pl.BlockSpec((1, H), lambda i: (0, 0)),
        ],
        out_specs=pl.BlockSpec((blk, H), lambda i: (i, 0)),
        out_shape=jax.ShapeDtypeStruct((E, H), F32),
    )(g, ea, we, w2, b2)


CHS = 40
NCS = EPW // CHS
NSL = 5


def _scatter_body(msg_hbm, dst_hbm, out_hbm, *s):
    idxs = s[0:NSL]
    mv = s[NSL:2 * NSL]
    zv = s[2 * NSL]
    accum = s[2 * NSL + 1]
    sli = s[2 * NSL + 2:2 * NSL + 2 + NSL]
    slm = s[2 * NSL + 2 + NSL:2 * NSL + 2 + 2 * NSL]
    sad = s[2 * NSL + 2 + 2 * NSL:2 * NSL + 2 + 3 * NSL]
    cid = lax.axis_index("c")
    sid = lax.axis_index("s")
    base = cid * (E // NC) + sid * EPW

    def off_of(t):
        return pl.multiple_of(base + t * CHS, 8)

    def zrow(r, c2):
        for k in range(8):
            zv[r, pl.ds(k * 16, 16)] = jnp.zeros((16,), F32)
        return c2

    lax.fori_loop(0, RCP, zrow, 0)

    nmine = (NRC - 1 - sid) // NS + 1

    def zcopy(j, c2):
        r0 = pl.multiple_of((sid + j * NS) * RCP, 8)
        pltpu.sync_copy(zv, accum.at[pl.ds(r0, RCP)])
        return c2

    lax.fori_loop(0, nmine, zcopy, 0)
    plsc.subcore_barrier()

    for b in range(NSL):
        off = off_of(b)
        pltpu.make_async_copy(dst_hbm.at[pl.ds(off, CHS)], idxs[b], sli[b]).start()
        pltpu.make_async_copy(msg_hbm.at[pl.ds(off, CHS)], mv[b], slm[b]).start()

    def outer(g, carry):
        for b in range(NSL):
            t = NSL * g + b
            off = off_of(t)
            pltpu.make_async_copy(dst_hbm.at[pl.ds(off, CHS)], idxs[b], sli[b]).wait()
            pltpu.make_async_copy(msg_hbm.at[pl.ds(off, CHS)], mv[b], slm[b]).wait()
            pltpu.async_copy(mv[b], accum.at[idxs[b]], sad[b], add=True)
        for b in range(NSL):
            t = NSL * g + b

            @pl.when(t + NSL < NCS)
            def _():
                noff = off_of(t + NSL)
                pltpu.make_async_copy(mv[b], accum.at[idxs[b]], sad[b]).wait()
                pltpu.make_async_copy(dst_hbm.at[pl.ds(noff, CHS)], idxs[b], sli[b]).start()
                pltpu.make_async_copy(msg_hbm.at[pl.ds(noff, CHS)], mv[b], slm[b]).start()
        return carry

    lax.fori_loop(0, NCS // NSL, outer, 0)
    for b in range(NSL):
        pltpu.make_async_copy(mv[b], accum.at[idxs[b]], sad[b]).wait()
    plsc.subcore_barrier()

    def ocopy(j, c2):
        r0 = pl.multiple_of((sid + j * NS) * RCP, 8)
        pltpu.sync_copy(accum.at[pl.ds(r0, RCP)],
                        out_hbm.at[pl.ds(pl.multiple_of(cid * N + r0, 8), RCP)])
        return c2

    lax.fori_loop(0, nmine, ocopy, 0)


_scatter = functools.partial(
    pl.kernel,
    out_type=jax.ShapeDtypeStruct((2 * N, H), F32),
    mesh=plsc.VectorSubcoreMesh(core_axis_name="c", subcore_axis_name="s",
                                num_cores=NC, num_subcores=NS),
    scratch_types=(
        [pltpu.VMEM((CHS,), jnp.int32) for _ in range(NSL)]
        + [pltpu.VMEM((CHS, H), F32) for _ in range(NSL)]
        + [pltpu.VMEM((RCP, H), F32), pltpu.VMEM_SHARED((N, H), F32)]
        + [pltpu.SemaphoreType.DMA for _ in range(3 * NSL)]
    ),
)(_scatter_body)


def _node_body(x_ref, p_ref, u1a_ref, u1b_ref, b1_ref, u2_ref, b2_ref,
               g_ref, bb_ref, o_ref):
    x = x_ref[...]
    aggr = p_ref[0, :, :] + p_ref[1, :, :]
    h = jnp.maximum(
        jnp.dot(x, u1a_ref[...], preferred_element_type=F32)
        + jnp.dot(aggr, u1b_ref[...], preferred_element_type=F32)
        + b1_ref[...], 0.0)
    o = jnp.maximum(jnp.dot(h, u2_ref[...], preferred_element_type=F32)
                    + b2_ref[...], 0.0)
    y = x + o
    mu = jnp.mean(y, axis=1, keepdims=True)
    var = jnp.mean((y - mu) * (y - mu), axis=1, keepdims=True)
    o_ref[...] = (y - mu) * lax.rsqrt(var + 1e-5) * g_ref[...] + bb_ref[...]


def _node(x, p, u1a, u1b, b1, u2, b2, g, b):
    blk = 1000
    grid = N // blk
    return pl.pallas_call(
        _node_body,
        grid=(grid,),
        in_specs=[
            pl.BlockSpec((blk, H), lambda i: (i, 0)),
            pl.BlockSpec((2, blk, H), lambda i: (0, i, 0)),
            pl.BlockSpec((H, H), lambda i: (0, 0)),
            pl.BlockSpec((H, H), lambda i: (0, 0)),
            pl.BlockSpec((1, H), lambda i: (0, 0)),
            pl.BlockSpec((H, H), lambda i: (0, 0)),
            pl.BlockSpec((1, H), lambda i: (0, 0)),
            pl.BlockSpec((1, H), lambda i: (0, 0)),
            pl.BlockSpec((1, H), lambda i: (0, 0)),
        ],
        out_specs=pl.BlockSpec((blk, H), lambda i: (i, 0)),
        out_shape=jax.ShapeDtypeStruct((N, H), F32),
    )(x, p, u1a, u1b, b1, u2, b2, g, b)


def kernel(x, edge_index, edge_attr, coords,
           msg_W1, msg_b1, msg_W2, msg_b2,
           upd_W1, upd_b1, upd_W2, upd_b2,
           ln_g, ln_b):
    src = edge_index[0]
    dst = edge_index[1]
    ws = msg_W1[:H]
    wd = msg_W1[H:2 * H]
    we = msg_W1[2 * H:2 * H + EF]
    wc = msg_W1[2 * H + EF:]
    coords_p = jnp.pad(coords, ((0, 0), (0, 5)))
    wc_p = jnp.pad(wc, ((0, 5), (0, 0)))

    a, b = _prep(x, coords_p, ws, wd, wc_p, msg_b1.reshape(1, H))
    g = _gather(a, b, src, dst)
    msg = _edge(g, edge_attr, we, msg_W2, msg_b2.reshape(1, H))
    p = _scatter(msg, dst).reshape(2, N, H)
    return _node(x, p, upd_W1[:H], upd_W1[H:], upd_b1.reshape(1, H),
                 upd_W2, upd_b2.reshape(1, H),
                 ln_g.reshape(1, H), ln_b.reshape(1, H))


# speedup vs baseline: 1.4324x; 1.1405x over previous
import functools

import jax
import jax.numpy as jnp
from jax import lax
from jax.experimental import pallas as pl
from jax.experimental.pallas import tpu as pltpu
from jax.experimental.pallas import tpu_sc as plsc

N = 10000
E = 320000
H = 128
EF = 16

NC = 2
NS = 16
NW = NC * NS
EPW = E // NW
CH = 80
NCHUNK = EPW // CH
RCP = 80
NRC = N // RCP
F32 = jnp.float32


def _prep_body(x_ref, c_ref, ws_ref, wd_ref, wc_ref, b1_ref, a_ref, b_ref):
    x = x_ref[...]
    cw = jnp.dot(c_ref[...], wc_ref[...], preferred_element_type=F32)
    a_ref[...] = jnp.dot(x, ws_ref[...], preferred_element_type=F32) - cw
    b_ref[...] = jnp.dot(x, wd_ref[...], preferred_element_type=F32) + cw + b1_ref[...]


def _prep(x, coords_p, ws, wd, wc_p, b1):
    blk = 1000
    grid = N // blk
    return pl.pallas_call(
        _prep_body,
        grid=(grid,),
        in_specs=[
            pl.BlockSpec((blk, H), lambda i: (i, 0)),
            pl.BlockSpec((blk, 8), lambda i: (i, 0)),
            pl.BlockSpec((H, H), lambda i: (0, 0)),
            pl.BlockSpec((H, H), lambda i: (0, 0)),
            pl.BlockSpec((8, H), lambda i: (0, 0)),
            pl.BlockSpec((1, H), lambda i: (0, 0)),
        ],
        out_specs=[
            pl.BlockSpec((blk, H), lambda i: (i, 0)),
            pl.BlockSpec((blk, H), lambda i: (i, 0)),
        ],
        out_shape=[
            jax.ShapeDtypeStruct((N, H), F32),
            jax.ShapeDtypeStruct((N, H), F32),
        ],
    )(x, coords_p, ws, wd, wc_p, b1)


CHG = 40
NSLG = 5


def _make_gather(ne):
    epw = ne // NW
    ncg = epw // CHG

    def _gather_body(a_hbm, b_hbm, src_hbm, dst_hbm, g_hbm, *s):
        isrc = s[0]
        idst = s[1]
        ra = s[2:2 + NSLG]
        rb = s[2 + NSLG:2 + 2 * NSLG]
        gv = s[2 + 2 * NSLG:2 + 3 * NSLG]
        sa = s[2 + 3 * NSLG:2 + 4 * NSLG]
        sb = s[2 + 4 * NSLG:2 + 5 * NSLG]
        sw = s[2 + 5 * NSLG:2 + 6 * NSLG]
        wid = lax.axis_index("s") * NC + lax.axis_index("c")
        base = wid * epw

        def off_of(t):
            return pl.multiple_of(base + t * CHG, 8)

        def sslice(t):
            return isrc.at[pl.ds(pl.multiple_of(t * CHG, 8), CHG)]

        def dslice(t):
            return idst.at[pl.ds(pl.multiple_of(t * CHG, 8), CHG)]

        pltpu.sync_copy(src_hbm.at[pl.ds(pl.multiple_of(base, 8), epw)], isrc)
        pltpu.sync_copy(dst_hbm.at[pl.ds(pl.multiple_of(base, 8), epw)], idst)
        for b in range(NSLG):
            pltpu.make_async_copy(a_hbm.at[sslice(b)], ra[b], sa[b]).start()
            pltpu.make_async_copy(b_hbm.at[dslice(b)], rb[b], sb[b]).start()

        def outer(q, carry):
            for b in range(NSLG):
                t = NSLG * q + b
                pltpu.make_async_copy(a_hbm.at[sslice(t)], ra[b], sa[b]).wait()
                pltpu.make_async_copy(b_hbm.at[dslice(t)], rb[b], sb[b]).wait()
                ra_b, rb_b, gv_b = ra[b], rb[b], gv[b]

                @pl.when(q > 0)
                def _():
                    pltpu.make_async_copy(gv_b, g_hbm.at[pl.ds(off_of(t - NSLG), CHG)],
                                          sw[b]).wait()

                def row(e, c2):
                    for k in range(8):
                        sl = pl.ds(k * 16, 16)
                        gv_b[e, sl] = ra_b[e, sl] + rb_b[e, sl]
                    return c2

                lax.fori_loop(0, CHG, row, 0, unroll=2)
                pltpu.make_async_copy(gv_b, g_hbm.at[pl.ds(off_of(t), CHG)], sw[b]).start()

                @pl.when(t + NSLG < ncg)
                def _():
                    pltpu.make_async_copy(a_hbm.at[sslice(t + NSLG)], ra[b], sa[b]).start()
                    pltpu.make_async_copy(b_hbm.at[dslice(t + NSLG)], rb[b], sb[b]).start()
            return carry

        lax.fori_loop(0, ncg // NSLG, outer, 0)
        for b in range(NSLG):
            pltpu.make_async_copy(gv[b], g_hbm.at[pl.ds(off_of(ncg - NSLG + b), CHG)],
                                  sw[b]).wait()

    return functools.partial(
        pl.kernel,
        out_type=jax.ShapeDtypeStruct((ne, H), F32),
        mesh=plsc.VectorSubcoreMesh(core_axis_name="c", subcore_axis_name="s",
                                    num_cores=NC, num_subcores=NS),
        scratch_types=(
            [pltpu.VMEM((epw,), jnp.int32) for _ in range(2)]
            + [pltpu.VMEM((CHG, H), F32) for _ in range(3 * NSLG)]
            + [pltpu.SemaphoreType.DMA for _ in range(3 * NSLG)]
        ),
    )(_gather_body)


_gather_half = _make_gather(E // 2)


def _edge_body(g_ref, ea_ref, we_ref, w2_ref, b2_ref, m_ref):
    h1 = jnp.maximum(
        g_ref[...] + jnp.dot(ea_ref[...], we_ref[...], preferred_element_type=F32),
        0.0)
    m_ref[...] = jnp.maximum(
        jnp.dot(h1, w2_ref[...], preferred_element_type=F32) + b2_ref[...],
        0.0)


def _edge(g, ea, we, w2, b2):
    ne = g.shape[0]
    blk = 3200
    grid = ne // blk
    return pl.pallas_call(
        _edge_body,
        grid=(grid,),
        in_specs=[
            pl.BlockSpec((blk, H), lambda i: (i, 0)),
            pl.BlockSpec((blk, EF), lambda i: (i, 0)),
            pl.BlockSpec((EF, H), lambda i: (0, 0)),
            pl.BlockSpec((H, H), lambda i: (0, 0)),
            pl.BlockSpec((1, H), lambda i: (0, 0)),
        ],
        out_specs=pl.BlockSpec((blk, H), lambda i: (i, 0)),
        out_shape=jax.ShapeDtypeStruct((ne, H), F32),
    )(g, ea, we, w2, b2)


CHS = 40
NSL = 5


def _make_scatter(ne):
    epw = ne // NW
    ncs = epw // CHS

    def _scatter_body(msg_hbm, dst_hbm, out_hbm, *s):
        idxs = s[0:NSL]
        mv = s[NSL:2 * NSL]
        zv = s[2 * NSL]
        accum = s[2 * NSL + 1]
        sli = s[2 * NSL + 2:2 * NSL + 2 + NSL]
        slm = s[2 * NSL + 2 + NSL:2 * NSL + 2 + 2 * NSL]
        sad = s[2 * NSL + 2 + 2 * NSL:2 * NSL + 2 + 3 * NSL]
        cid = lax.axis_index("c")
        sid = lax.axis_index("s")
        base = cid * (ne // NC) + sid * epw

        def off_of(t):
            return pl.multiple_of(base + t * CHS, 8)

        def zrow(r, c2):
            for k in range(8):
                zv[r, pl.ds(k * 16, 16)] = jnp.zeros((16,), F32)
            return c2

        lax.fori_loop(0, RCP, zrow, 0)

        nmine = (NRC - 1 - sid) // NS + 1

        def zcopy(j, c2):
            r0 = pl.multiple_of((sid + j * NS) * RCP, 8)
            pltpu.sync_copy(zv, accum.at[pl.ds(r0, RCP)])
            return c2

        lax.fori_loop(0, nmine, zcopy, 0)
        plsc.subcore_barrier()

        for b in range(NSL):
            off = off_of(b)
            pltpu.make_async_copy(dst_hbm.at[pl.ds(off, CHS)], idxs[b], sli[b]).start()
            pltpu.make_async_copy(msg_hbm.at[pl.ds(off, CHS)], mv[b], slm[b]).start()

        def outer(g, carry):
            for b in range(NSL):
                t = NSL * g + b
                off = off_of(t)
                pltpu.make_async_copy(dst_hbm.at[pl.ds(off, CHS)], idxs[b], sli[b]).wait()
                pltpu.make_async_copy(msg_hbm.at[pl.ds(off, CHS)], mv[b], slm[b]).wait()
                pltpu.async_copy(mv[b], accum.at[idxs[b]], sad[b], add=True)
            for b in range(NSL):
                t = NSL * g + b

                @pl.when(t + NSL < ncs)
                def _():
                    noff = off_of(t + NSL)
                    pltpu.make_async_copy(mv[b], accum.at[idxs[b]], sad[b]).wait()
                    pltpu.make_async_copy(dst_hbm.at[pl.ds(noff, CHS)], idxs[b], sli[b]).start()
                    pltpu.make_async_copy(msg_hbm.at[pl.ds(noff, CHS)], mv[b], slm[b]).start()
            return carry

        lax.fori_loop(0, ncs // NSL, outer, 0)
        for b in range(NSL):
            pltpu.make_async_copy(mv[b], accum.at[idxs[b]], sad[b]).wait()
        plsc.subcore_barrier()

        def ocopy(j, c2):
            r0 = pl.multiple_of((sid + j * NS) * RCP, 8)
            pltpu.sync_copy(accum.at[pl.ds(r0, RCP)],
                            out_hbm.at[pl.ds(pl.multiple_of(cid * N + r0, 8), RCP)])
            return c2

        lax.fori_loop(0, nmine, ocopy, 0)

    return functools.partial(
        pl.kernel,
        out_type=jax.ShapeDtypeStruct((2 * N, H), F32),
        mesh=plsc.VectorSubcoreMesh(core_axis_name="c", subcore_axis_name="s",
                                    num_cores=NC, num_subcores=NS),
        scratch_types=(
            [pltpu.VMEM((CHS,), jnp.int32) for _ in range(NSL)]
            + [pltpu.VMEM((CHS, H), F32) for _ in range(NSL)]
            + [pltpu.VMEM((RCP, H), F32), pltpu.VMEM_SHARED((N, H), F32)]
            + [pltpu.SemaphoreType.DMA for _ in range(3 * NSL)]
        ),
    )(_scatter_body)


_scatter_half = _make_scatter(E // 2)


def _node_body(x_ref, p_ref, q_ref, u1a_ref, u1b_ref, b1_ref, u2_ref, b2_ref,
               g_ref, bb_ref, o_ref):
    x = x_ref[...]
    aggr = (p_ref[0, :, :] + p_ref[1, :, :]) + (q_ref[0, :, :] + q_ref[1, :, :])
    h = jnp.maximum(
        jnp.dot(x, u1a_ref[...], preferred_element_type=F32)
        + jnp.dot(aggr, u1b_ref[...], preferred_element_type=F32)
        + b1_ref[...], 0.0)
    o = jnp.maximum(jnp.dot(h, u2_ref[...], preferred_element_type=F32)
                    + b2_ref[...], 0.0)
    y = x + o
    mu = jnp.mean(y, axis=1, keepdims=True)
    var = jnp.mean((y - mu) * (y - mu), axis=1, keepdims=True)
    o_ref[...] = (y - mu) * lax.rsqrt(var + 1e-5) * g_ref[...] + bb_ref[...]


def _node(x, p, q, u1a, u1b, b1, u2, b2, g, b):
    blk = 1000
    grid = N // blk
    return pl.pallas_call(
        _node_body,
        grid=(grid,),
        in_specs=[
            pl.BlockSpec((blk, H), lambda i: (i, 0)),
            pl.BlockSpec((2, blk, H), lambda i: (0, i, 0)),
            pl.BlockSpec((2, blk, H), lambda i: (0, i, 0)),
            pl.BlockSpec((H, H), lambda i: (0, 0)),
            pl.BlockSpec((H, H), lambda i: (0, 0)),
            pl.BlockSpec((1, H), lambda i: (0, 0)),
            pl.BlockSpec((H, H), lambda i: (0, 0)),
            pl.BlockSpec((1, H), lambda i: (0, 0)),
            pl.BlockSpec((1, H), lambda i: (0, 0)),
            pl.BlockSpec((1, H), lambda i: (0, 0)),
        ],
        out_specs=pl.BlockSpec((blk, H), lambda i: (i, 0)),
        out_shape=jax.ShapeDtypeStruct((N, H), F32),
    )(x, p, q, u1a, u1b, b1, u2, b2, g, b)


def kernel(x, edge_index, edge_attr, coords,
           msg_W1, msg_b1, msg_W2, msg_b2,
           upd_W1, upd_b1, upd_W2, upd_b2,
           ln_g, ln_b):
    src = edge_index[0]
    dst = edge_index[1]
    ws = msg_W1[:H]
    wd = msg_W1[H:2 * H]
    we = msg_W1[2 * H:2 * H + EF]
    wc = msg_W1[2 * H + EF:]
    coords_p = jnp.pad(coords, ((0, 0), (0, 5)))
    wc_p = jnp.pad(wc, ((0, 5), (0, 0)))

    a, b = _prep(x, coords_p, ws, wd, wc_p, msg_b1.reshape(1, H))
    eh = E // 2
    src1, src2 = src[:eh], src[eh:]
    dst1, dst2 = dst[:eh], dst[eh:]
    ea1, ea2 = edge_attr[:eh], edge_attr[eh:]
    b2r = msg_b2.reshape(1, H)
    g1 = _gather_half(a, b, src1, dst1)
    msg1 = _edge(g1, ea1, we, msg_W2, b2r)
    g2 = _gather_half(a, b, src2, dst2)
    msg2 = _edge(g2, ea2, we, msg_W2, b2r)
    p = _scatter_half(msg1, dst1).reshape(2, N, H)
    q = _scatter_half(msg2, dst2).reshape(2, N, H)
    return _node(x, p, q, upd_W1[:H], upd_W1[H:], upd_b1.reshape(1, H),
                 upd_W2, upd_b2.reshape(1, H),
                 ln_g.reshape(1, H), ln_b.reshape(1, H))


# speedup vs baseline: 1.7152x; 1.1974x over previous
import functools

import jax
import jax.numpy as jnp
from jax import lax
from jax.experimental import pallas as pl
from jax.experimental.pallas import tpu as pltpu
from jax.experimental.pallas import tpu_sc as plsc

N = 10000
E = 320000
H = 128
EF = 16

NC = 2
NS = 16
NW = NC * NS
EPW = E // NW
CH = 80
NCHUNK = EPW // CH
RCP = 80
NRC = N // RCP
F32 = jnp.float32


def _prep_body(x_ref, c_ref, ws_ref, wd_ref, wc_ref, b1_ref, a_ref, b_ref):
    x = x_ref[...]
    cw = jnp.dot(c_ref[...], wc_ref[...], preferred_element_type=F32)
    a_ref[...] = jnp.dot(x, ws_ref[...], preferred_element_type=F32) - cw
    b_ref[...] = jnp.dot(x, wd_ref[...], preferred_element_type=F32) + cw + b1_ref[...]


def _prep(x, coords_p, ws, wd, wc_p, b1):
    blk = 1000
    grid = N // blk
    return pl.pallas_call(
        _prep_body,
        grid=(grid,),
        in_specs=[
            pl.BlockSpec((blk, H), lambda i: (i, 0)),
            pl.BlockSpec((blk, 8), lambda i: (i, 0)),
            pl.BlockSpec((H, H), lambda i: (0, 0)),
            pl.BlockSpec((H, H), lambda i: (0, 0)),
            pl.BlockSpec((8, H), lambda i: (0, 0)),
            pl.BlockSpec((1, H), lambda i: (0, 0)),
        ],
        out_specs=[
            pl.BlockSpec((blk, H), lambda i: (i, 0)),
            pl.BlockSpec((blk, H), lambda i: (i, 0)),
        ],
        out_shape=[
            jax.ShapeDtypeStruct((N, H), F32),
            jax.ShapeDtypeStruct((N, H), F32),
        ],
    )(x, coords_p, ws, wd, wc_p, b1)


CHG = 40
NSLG = 5


def _make_gather2(ne):
    ept = ne // NS
    ncg = ept // CHG

    def _gather_body(a_hbm, b_hbm, src_hbm, dst_hbm, ga_hbm, gb_hbm, *s):
        idxv = s[0]
        rows = s[1:1 + NSLG]
        tbl = s[1 + NSLG]
        sg = s[2 + NSLG:2 + 2 * NSLG]
        sw = s[2 + 2 * NSLG:2 + 3 * NSLG]
        cid = lax.axis_index("c")
        sid = lax.axis_index("s")
        base = sid * ept

        nmine = (NRC - 1 - sid) // NS + 1

        def tload(j, c2):
            r0 = pl.multiple_of((sid + j * NS) * RCP, 8)

            @pl.when(cid == 0)
            def _():
                pltpu.sync_copy(a_hbm.at[pl.ds(r0, RCP)], tbl.at[pl.ds(r0, RCP)])

            @pl.when(cid == 1)
            def _():
                pltpu.sync_copy(b_hbm.at[pl.ds(r0, RCP)], tbl.at[pl.ds(r0, RCP)])

            return c2

        lax.fori_loop(0, nmine, tload, 0)

        @pl.when(cid == 0)
        def _():
            pltpu.sync_copy(src_hbm.at[pl.ds(pl.multiple_of(base, 8), ept)], idxv)

        @pl.when(cid == 1)
        def _():
            pltpu.sync_copy(dst_hbm.at[pl.ds(pl.multiple_of(base, 8), ept)], idxv)

        plsc.subcore_barrier()

        def off_of(t):
            return pl.multiple_of(base + t * CHG, 8)

        def islice(t):
            return idxv.at[pl.ds(pl.multiple_of(t * CHG, 8), CHG)]

        for b in range(NSLG):
            pltpu.make_async_copy(tbl.at[islice(b)], rows[b], sg[b]).start()

        def outer(q, carry):
            for b in range(NSLG):
                t = NSLG * q + b
                pltpu.make_async_copy(tbl.at[islice(t)], rows[b], sg[b]).wait()
                r_b = rows[b]
                o = pl.ds(off_of(t), CHG)

                @pl.when(cid == 0)
                def _():
                    pltpu.make_async_copy(r_b, ga_hbm.at[o], sw[b]).start()

                @pl.when(cid == 1)
                def _():
                    pltpu.make_async_copy(r_b, gb_hbm.at[o], sw[b]).start()

            for b in range(NSLG):
                t = NSLG * q + b
                r_b = rows[b]
                o = pl.ds(off_of(t), CHG)

                @pl.when(t + NSLG < ncg)
                def _():
                    @pl.when(cid == 0)
                    def _():
                        pltpu.make_async_copy(r_b, ga_hbm.at[o], sw[b]).wait()

                    @pl.when(cid == 1)
                    def _():
                        pltpu.make_async_copy(r_b, gb_hbm.at[o], sw[b]).wait()

                    pltpu.make_async_copy(tbl.at[islice(t + NSLG)], rows[b],
                                          sg[b]).start()

            return carry

        lax.fori_loop(0, ncg // NSLG, outer, 0)
        for b in range(NSLG):
            t = ncg - NSLG + b
            r_b = rows[b]
            o = pl.ds(off_of(t), CHG)

            @pl.when(cid == 0)
            def _():
                pltpu.make_async_copy(r_b, ga_hbm.at[o], sw[b]).wait()

            @pl.when(cid == 1)
            def _():
                pltpu.make_async_copy(r_b, gb_hbm.at[o], sw[b]).wait()

    return functools.partial(
        pl.kernel,
        out_type=[jax.ShapeDtypeStruct((ne, H), F32),
                  jax.ShapeDtypeStruct((ne, H), F32)],
        mesh=plsc.VectorSubcoreMesh(core_axis_name="c", subcore_axis_name="s",
                                    num_cores=NC, num_subcores=NS),
        scratch_types=(
            [pltpu.VMEM((ept,), jnp.int32)]
            + [pltpu.VMEM((CHG, H), F32) for _ in range(NSLG)]
            + [pltpu.VMEM_SHARED((N, H), F32)]
            + [pltpu.SemaphoreType.DMA for _ in range(2 * NSLG)]
        ),
    )(_gather_body)


_gather2_half = _make_gather2(E // 2)


def _edge_body(ga_ref, gb_ref, ea_ref, we_ref, w2_ref, b2_ref, m_ref):
    h1 = jnp.maximum(
        ga_ref[...] + gb_ref[...]
        + jnp.dot(ea_ref[...], we_ref[...], preferred_element_type=F32),
        0.0)
    m_ref[...] = jnp.maximum(
        jnp.dot(h1, w2_ref[...], preferred_element_type=F32) + b2_ref[...],
        0.0)


def _edge(ga, gb, ea, we, w2, b2):
    ne = ga.shape[0]
    blk = 3200
    grid = ne // blk
    return pl.pallas_call(
        _edge_body,
        grid=(grid,),
        in_specs=[
            pl.BlockSpec((blk, H), lambda i: (i, 0)),
            pl.BlockSpec((blk, H), lambda i: (i, 0)),
            pl.BlockSpec((blk, EF), lambda i: (i, 0)),
            pl.BlockSpec((EF, H), lambda i: (0, 0)),
            pl.BlockSpec((H, H), lambda i: (0, 0)),
            pl.BlockSpec((1, H), lambda i: (0, 0)),
        ],
        out_specs=pl.BlockSpec((blk, H), lambda i: (i, 0)),
        out_shape=jax.ShapeDtypeStruct((ne, H), F32),
    )(ga, gb, ea, we, w2, b2)


CHS = 40
NSL = 5


def _make_scatter(ne):
    epw = ne // NW
    ncs = epw // CHS

    def _scatter_body(msg_hbm, dst_hbm, out_hbm, *s):
        idxs = s[0:NSL]
        mv = s[NSL:2 * NSL]
        zv = s[2 * NSL]
        accum = s[2 * NSL + 1]
        sli = s[2 * NSL + 2:2 * NSL + 2 + NSL]
        slm = s[2 * NSL + 2 + NSL:2 * NSL + 2 + 2 * NSL]
        sad = s[2 * NSL + 2 + 2 * NSL:2 * NSL + 2 + 3 * NSL]
        cid = lax.axis_index("c")
        sid = lax.axis_index("s")
        base = cid * (ne // NC) + sid * epw

        def off_of(t):
            return pl.multiple_of(base + t * CHS, 8)

        def zrow(r, c2):
            for k in range(8):
                zv[r, pl.ds(k * 16, 16)] = jnp.zeros((16,), F32)
            return c2

        lax.fori_loop(0, RCP, zrow, 0)

        nmine = (NRC - 1 - sid) // NS + 1

        def zcopy(j, c2):
            r0 = pl.multiple_of((sid + j * NS) * RCP, 8)
            pltpu.sync_copy(zv, accum.at[pl.ds(r0, RCP)])
            return c2

        lax.fori_loop(0, nmine, zcopy, 0)
        plsc.subcore_barrier()

        for b in range(NSL):
            off = off_of(b)
            pltpu.make_async_copy(dst_hbm.at[pl.ds(off, CHS)], idxs[b], sli[b]).start()
            pltpu.make_async_copy(msg_hbm.at[pl.ds(off, CHS)], mv[b], slm[b]).start()

        def outer(g, carry):
            for b in range(NSL):
                t = NSL * g + b
                off = off_of(t)
                pltpu.make_async_copy(dst_hbm.at[pl.ds(off, CHS)], idxs[b], sli[b]).wait()
                pltpu.make_async_copy(msg_hbm.at[pl.ds(off, CHS)], mv[b], slm[b]).wait()
                pltpu.async_copy(mv[b], accum.at[idxs[b]], sad[b], add=True)
            for b in range(NSL):
                t = NSL * g + b

                @pl.when(t + NSL < ncs)
                def _():
                    noff = off_of(t + NSL)
                    pltpu.make_async_copy(mv[b], accum.at[idxs[b]], sad[b]).wait()
                    pltpu.make_async_copy(dst_hbm.at[pl.ds(noff, CHS)], idxs[b], sli[b]).start()
                    pltpu.make_async_copy(msg_hbm.at[pl.ds(noff, CHS)], mv[b], slm[b]).start()
            return carry

        lax.fori_loop(0, ncs // NSL, outer, 0)
        for b in range(NSL):
            pltpu.make_async_copy(mv[b], accum.at[idxs[b]], sad[b]).wait()
        plsc.subcore_barrier()

        def ocopy(j, c2):
            r0 = pl.multiple_of((sid + j * NS) * RCP, 8)
            pltpu.sync_copy(accum.at[pl.ds(r0, RCP)],
                            out_hbm.at[pl.ds(pl.multiple_of(cid * N + r0, 8), RCP)])
            return c2

        lax.fori_loop(0, nmine, ocopy, 0)

    return functools.partial(
        pl.kernel,
        out_type=jax.ShapeDtypeStruct((2 * N, H), F32),
        mesh=plsc.VectorSubcoreMesh(core_axis_name="c", subcore_axis_name="s",
                                    num_cores=NC, num_subcores=NS),
        scratch_types=(
            [pltpu.VMEM((CHS,), jnp.int32) for _ in range(NSL)]
            + [pltpu.VMEM((CHS, H), F32) for _ in range(NSL)]
            + [pltpu.VMEM((RCP, H), F32), pltpu.VMEM_SHARED((N, H), F32)]
            + [pltpu.SemaphoreType.DMA for _ in range(3 * NSL)]
        ),
    )(_scatter_body)


_scatter_half = _make_scatter(E // 2)


def _node_body(x_ref, p_ref, q_ref, u1a_ref, u1b_ref, b1_ref, u2_ref, b2_ref,
               g_ref, bb_ref, o_ref):
    x = x_ref[...]
    aggr = (p_ref[0, :, :] + p_ref[1, :, :]) + (q_ref[0, :, :] + q_ref[1, :, :])
    h = jnp.maximum(
        jnp.dot(x, u1a_ref[...], preferred_element_type=F32)
        + jnp.dot(aggr, u1b_ref[...], preferred_element_type=F32)
        + b1_ref[...], 0.0)
    o = jnp.maximum(jnp.dot(h, u2_ref[...], preferred_element_type=F32)
                    + b2_ref[...], 0.0)
    y = x + o
    mu = jnp.mean(y, axis=1, keepdims=True)
    var = jnp.mean((y - mu) * (y - mu), axis=1, keepdims=True)
    o_ref[...] = (y - mu) * lax.rsqrt(var + 1e-5) * g_ref[...] + bb_ref[...]


def _node(x, p, q, u1a, u1b, b1, u2, b2, g, b):
    blk = 1000
    grid = N // blk
    return pl.pallas_call(
        _node_body,
        grid=(grid,),
        in_specs=[
            pl.BlockSpec((blk, H), lambda i: (i, 0)),
            pl.BlockSpec((2, blk, H), lambda i: (0, i, 0)),
            pl.BlockSpec((2, blk, H), lambda i: (0, i, 0)),
            pl.BlockSpec((H, H), lambda i: (0, 0)),
            pl.BlockSpec((H, H), lambda i: (0, 0)),
            pl.BlockSpec((1, H), lambda i: (0, 0)),
            pl.BlockSpec((H, H), lambda i: (0, 0)),
            pl.BlockSpec((1, H), lambda i: (0, 0)),
            pl.BlockSpec((1, H), lambda i: (0, 0)),
            pl.BlockSpec((1, H), lambda i: (0, 0)),
        ],
        out_specs=pl.BlockSpec((blk, H), lambda i: (i, 0)),
        out_shape=jax.ShapeDtypeStruct((N, H), F32),
    )(x, p, q, u1a, u1b, b1, u2, b2, g, b)


def kernel(x, edge_index, edge_attr, coords,
           msg_W1, msg_b1, msg_W2, msg_b2,
           upd_W1, upd_b1, upd_W2, upd_b2,
           ln_g, ln_b):
    src = edge_index[0]
    dst = edge_index[1]
    ws = msg_W1[:H]
    wd = msg_W1[H:2 * H]
    we = msg_W1[2 * H:2 * H + EF]
    wc = msg_W1[2 * H + EF:]
    coords_p = jnp.pad(coords, ((0, 0), (0, 5)))
    wc_p = jnp.pad(wc, ((0, 5), (0, 0)))

    a, b = _prep(x, coords_p, ws, wd, wc_p, msg_b1.reshape(1, H))
    eh = E // 2
    src1, src2 = src[:eh], src[eh:]
    dst1, dst2 = dst[:eh], dst[eh:]
    ea1, ea2 = edge_attr[:eh], edge_attr[eh:]
    b2r = msg_b2.reshape(1, H)
    ga1, gb1 = _gather2_half(a, b, src1, dst1)
    msg1 = _edge(ga1, gb1, ea1, we, msg_W2, b2r)
    ga2, gb2 = _gather2_half(a, b, src2, dst2)
    msg2 = _edge(ga2, gb2, ea2, we, msg_W2, b2r)
    p = _scatter_half(msg1, dst1).reshape(2, N, H)
    q = _scatter_half(msg2, dst2).reshape(2, N, H)
    return _node(x, p, q, upd_W1[:H], upd_W1[H:], upd_b1.reshape(1, H),
                 upd_W2, upd_b2.reshape(1, H),
                 ln_g.reshape(1, H), ln_b.reshape(1, H))


# speedup vs baseline: 1.7867x; 1.0417x over previous
import functools

import jax
import jax.numpy as jnp
from jax import lax
from jax.experimental import pallas as pl
from jax.experimental.pallas import tpu as pltpu
from jax.experimental.pallas import tpu_sc as plsc

N = 10000
E = 320000
H = 128
EF = 16

NC = 2
NS = 16
NW = NC * NS
EPW = E // NW
CH = 80
NCHUNK = EPW // CH
RCP = 80
NRC = N // RCP
F32 = jnp.float32


def _prep_body(x_ref, c_ref, ws_ref, wd_ref, wc_ref, b1_ref, a_ref, b_ref):
    x = x_ref[...]
    cw = jnp.dot(c_ref[...], wc_ref[...], preferred_element_type=F32)
    a_ref[...] = jnp.dot(x, ws_ref[...], preferred_element_type=F32) - cw
    b_ref[...] = jnp.dot(x, wd_ref[...], preferred_element_type=F32) + cw + b1_ref[...]


def _prep(x, coords_p, ws, wd, wc_p, b1):
    blk = 1000
    grid = N // blk
    return pl.pallas_call(
        _prep_body,
        grid=(grid,),
        in_specs=[
            pl.BlockSpec((blk, H), lambda i: (i, 0)),
            pl.BlockSpec((blk, 8), lambda i: (i, 0)),
            pl.BlockSpec((H, H), lambda i: (0, 0)),
            pl.BlockSpec((H, H), lambda i: (0, 0)),
            pl.BlockSpec((8, H), lambda i: (0, 0)),
            pl.BlockSpec((1, H), lambda i: (0, 0)),
        ],
        out_specs=[
            pl.BlockSpec((blk, H), lambda i: (i, 0)),
            pl.BlockSpec((blk, H), lambda i: (i, 0)),
        ],
        out_shape=[
            jax.ShapeDtypeStruct((N, H), F32),
            jax.ShapeDtypeStruct((N, H), F32),
        ],
    )(x, coords_p, ws, wd, wc_p, b1)


CHG = 40
NSLG = 5


def _make_gather2(ne, eoff):
    ept = ne // NS
    ncg = ept // CHG

    def _gather_body(a_hbm, b_hbm, src_hbm, dst_hbm, ga_hbm, gb_hbm, *s):
        idxv = s[0]
        rows = s[1:1 + NSLG]
        tbl = s[1 + NSLG]
        sg = s[2 + NSLG:2 + 2 * NSLG]
        sw = s[2 + 2 * NSLG:2 + 3 * NSLG]
        stl = s[2 + 3 * NSLG]
        cid = lax.axis_index("c")
        sid = lax.axis_index("s")
        base = sid * ept

        nmine = (NRC - 1 - sid) // NS + 1

        def tload(j, c2):
            r0 = pl.multiple_of((sid + j * NS) * RCP, 8)

            @pl.when(cid == 0)
            def _():
                pltpu.make_async_copy(a_hbm.at[pl.ds(r0, RCP)],
                                      tbl.at[pl.ds(r0, RCP)], stl).start()

            @pl.when(cid == 1)
            def _():
                pltpu.make_async_copy(b_hbm.at[pl.ds(r0, RCP)],
                                      tbl.at[pl.ds(r0, RCP)], stl).start()

            return c2

        lax.fori_loop(0, nmine, tload, 0)

        @pl.when(cid == 0)
        def _():
            pltpu.sync_copy(src_hbm.at[pl.ds(pl.multiple_of(eoff + base, 8), ept)],
                            idxv)

        @pl.when(cid == 1)
        def _():
            pltpu.sync_copy(dst_hbm.at[pl.ds(pl.multiple_of(eoff + base, 8), ept)],
                            idxv)

        def twait(j, c2):
            r0 = pl.multiple_of((sid + j * NS) * RCP, 8)
            pltpu.make_async_copy(a_hbm.at[pl.ds(r0, RCP)],
                                  tbl.at[pl.ds(r0, RCP)], stl).wait()
            return c2

        lax.fori_loop(0, nmine, twait, 0)
        plsc.subcore_barrier()

        def off_of(t):
            return pl.multiple_of(base + t * CHG, 8)

        def islice(t):
            return idxv.at[pl.ds(pl.multiple_of(t * CHG, 8), CHG)]

        for b in range(NSLG):
            pltpu.make_async_copy(tbl.at[islice(b)], rows[b], sg[b]).start()

        def outer(q, carry):
            for b in range(NSLG):
                t = NSLG * q + b
                pltpu.make_async_copy(tbl.at[islice(t)], rows[b], sg[b]).wait()
                r_b = rows[b]
                o = pl.ds(off_of(t), CHG)

                @pl.when(cid == 0)
                def _():
                    pltpu.make_async_copy(r_b, ga_hbm.at[o], sw[b]).start()

                @pl.when(cid == 1)
                def _():
                    pltpu.make_async_copy(r_b, gb_hbm.at[o], sw[b]).start()

            for b in range(NSLG):
                t = NSLG * q + b
                r_b = rows[b]
                o = pl.ds(off_of(t), CHG)

                @pl.when(t + NSLG < ncg)
                def _():
                    @pl.when(cid == 0)
                    def _():
                        pltpu.make_async_copy(r_b, ga_hbm.at[o], sw[b]).wait()

                    @pl.when(cid == 1)
                    def _():
                        pltpu.make_async_copy(r_b, gb_hbm.at[o], sw[b]).wait()

                    pltpu.make_async_copy(tbl.at[islice(t + NSLG)], rows[b],
                                          sg[b]).start()

            return carry

        lax.fori_loop(0, ncg // NSLG, outer, 0)
        for b in range(NSLG):
            t = ncg - NSLG + b
            r_b = rows[b]
            o = pl.ds(off_of(t), CHG)

            @pl.when(cid == 0)
            def _():
                pltpu.make_async_copy(r_b, ga_hbm.at[o], sw[b]).wait()

            @pl.when(cid == 1)
            def _():
                pltpu.make_async_copy(r_b, gb_hbm.at[o], sw[b]).wait()

    return functools.partial(
        pl.kernel,
        out_type=[jax.ShapeDtypeStruct((ne, H), F32),
                  jax.ShapeDtypeStruct((ne, H), F32)],
        mesh=plsc.VectorSubcoreMesh(core_axis_name="c", subcore_axis_name="s",
                                    num_cores=NC, num_subcores=NS),
        scratch_types=(
            [pltpu.VMEM((ept,), jnp.int32)]
            + [pltpu.VMEM((CHG, H), F32) for _ in range(NSLG)]
            + [pltpu.VMEM_SHARED((N, H), F32)]
            + [pltpu.SemaphoreType.DMA for _ in range(2 * NSLG + 1)]
        ),
    )(_gather_body)


_gather2_h1 = _make_gather2(E // 2, 0)
_gather2_h2 = _make_gather2(E // 2, E // 2)


def _edge_body(ga_ref, gb_ref, ea_ref, we_ref, w2_ref, b2_ref, m_ref):
    h1 = jnp.maximum(
        ga_ref[...] + gb_ref[...]
        + jnp.dot(ea_ref[...], we_ref[...], preferred_element_type=F32),
        0.0)
    m_ref[...] = jnp.maximum(
        jnp.dot(h1, w2_ref[...], preferred_element_type=F32) + b2_ref[...],
        0.0)


def _edge(ga, gb, ea, we, w2, b2, boff):
    ne = ga.shape[0]
    blk = 3200
    grid = ne // blk
    return pl.pallas_call(
        _edge_body,
        grid=(grid,),
        in_specs=[
            pl.BlockSpec((blk, H), lambda i: (i, 0)),
            pl.BlockSpec((blk, H), lambda i: (i, 0)),
            pl.BlockSpec((blk, EF), lambda i: (i + boff, 0)),
            pl.BlockSpec((EF, H), lambda i: (0, 0)),
            pl.BlockSpec((H, H), lambda i: (0, 0)),
            pl.BlockSpec((1, H), lambda i: (0, 0)),
        ],
        out_specs=pl.BlockSpec((blk, H), lambda i: (i, 0)),
        out_shape=jax.ShapeDtypeStruct((ne, H), F32),
    )(ga, gb, ea, we, w2, b2)


CHS = 40
NSL = 5


def _make_scatter(ne, eoff):
    epw = ne // NW
    ncs = epw // CHS

    def _scatter_body(msg_hbm, dst_hbm, out_hbm, *s):
        idxs = s[0:NSL]
        mv = s[NSL:2 * NSL]
        zv = s[2 * NSL]
        accum = s[2 * NSL + 1]
        sli = s[2 * NSL + 2:2 * NSL + 2 + NSL]
        slm = s[2 * NSL + 2 + NSL:2 * NSL + 2 + 2 * NSL]
        sad = s[2 * NSL + 2 + 2 * NSL:2 * NSL + 2 + 3 * NSL]
        cid = lax.axis_index("c")
        sid = lax.axis_index("s")
        base = cid * (ne // NC) + sid * epw

        def off_of(t):
            return pl.multiple_of(base + t * CHS, 8)

        def doff_of(t):
            return pl.multiple_of(eoff + base + t * CHS, 8)

        def zrow(r, c2):
            for k in range(8):
                zv[r, pl.ds(k * 16, 16)] = jnp.zeros((16,), F32)
            return c2

        lax.fori_loop(0, RCP, zrow, 0)

        nmine = (NRC - 1 - sid) // NS + 1

        def zcopy(j, c2):
            r0 = pl.multiple_of((sid + j * NS) * RCP, 8)
            pltpu.sync_copy(zv, accum.at[pl.ds(r0, RCP)])
            return c2

        lax.fori_loop(0, nmine, zcopy, 0)
        plsc.subcore_barrier()

        for b in range(NSL):
            off = off_of(b)
            pltpu.make_async_copy(dst_hbm.at[pl.ds(doff_of(b), CHS)], idxs[b], sli[b]).start()
            pltpu.make_async_copy(msg_hbm.at[pl.ds(off, CHS)], mv[b], slm[b]).start()

        def outer(g, carry):
            for b in range(NSL):
                t = NSL * g + b
                off = off_of(t)
                pltpu.make_async_copy(dst_hbm.at[pl.ds(doff_of(t), CHS)], idxs[b], sli[b]).wait()
                pltpu.make_async_copy(msg_hbm.at[pl.ds(off, CHS)], mv[b], slm[b]).wait()
                pltpu.async_copy(mv[b], accum.at[idxs[b]], sad[b], add=True)
            for b in range(NSL):
                t = NSL * g + b

                @pl.when(t + NSL < ncs)
                def _():
                    noff = off_of(t + NSL)
                    pltpu.make_async_copy(mv[b], accum.at[idxs[b]], sad[b]).wait()
                    pltpu.make_async_copy(dst_hbm.at[pl.ds(doff_of(t + NSL), CHS)], idxs[b], sli[b]).start()
                    pltpu.make_async_copy(msg_hbm.at[pl.ds(noff, CHS)], mv[b], slm[b]).start()
            return carry

        lax.fori_loop(0, ncs // NSL, outer, 0)
        for b in range(NSL):
            pltpu.make_async_copy(mv[b], accum.at[idxs[b]], sad[b]).wait()
        plsc.subcore_barrier()

        def ocopy(j, c2):
            r0 = pl.multiple_of((sid + j * NS) * RCP, 8)
            pltpu.sync_copy(accum.at[pl.ds(r0, RCP)],
                            out_hbm.at[pl.ds(pl.multiple_of(cid * N + r0, 8), RCP)])
            return c2

        lax.fori_loop(0, nmine, ocopy, 0)

    return functools.partial(
        pl.kernel,
        out_type=jax.ShapeDtypeStruct((2 * N, H), F32),
        mesh=plsc.VectorSubcoreMesh(core_axis_name="c", subcore_axis_name="s",
                                    num_cores=NC, num_subcores=NS),
        scratch_types=(
            [pltpu.VMEM((CHS,), jnp.int32) for _ in range(NSL)]
            + [pltpu.VMEM((CHS, H), F32) for _ in range(NSL)]
            + [pltpu.VMEM((RCP, H), F32), pltpu.VMEM_SHARED((N, H), F32)]
            + [pltpu.SemaphoreType.DMA for _ in range(3 * NSL)]
        ),
    )(_scatter_body)


_scatter_h1 = _make_scatter(E // 2, 0)
_scatter_h2 = _make_scatter(E // 2, E // 2)


def _node_body(x_ref, p_ref, q_ref, u1a_ref, u1b_ref, b1_ref, u2_ref, b2_ref,
               g_ref, bb_ref, o_ref):
    x = x_ref[...]
    aggr = (p_ref[0, :, :] + p_ref[1, :, :]) + (q_ref[0, :, :] + q_ref[1, :, :])
    h = jnp.maximum(
        jnp.dot(x, u1a_ref[...], preferred_element_type=F32)
        + jnp.dot(aggr, u1b_ref[...], preferred_element_type=F32)
        + b1_ref[...], 0.0)
    o = jnp.maximum(jnp.dot(h, u2_ref[...], preferred_element_type=F32)
                    + b2_ref[...], 0.0)
    y = x + o
    mu = jnp.mean(y, axis=1, keepdims=True)
    var = jnp.mean((y - mu) * (y - mu), axis=1, keepdims=True)
    o_ref[...] = (y - mu) * lax.rsqrt(var + 1e-5) * g_ref[...] + bb_ref[...]


def _node(x, p, q, u1a, u1b, b1, u2, b2, g, b):
    blk = 1000
    grid = N // blk
    return pl.pallas_call(
        _node_body,
        grid=(grid,),
        in_specs=[
            pl.BlockSpec((blk, H), lambda i: (i, 0)),
            pl.BlockSpec((2, blk, H), lambda i: (0, i, 0)),
            pl.BlockSpec((2, blk, H), lambda i: (0, i, 0)),
            pl.BlockSpec((H, H), lambda i: (0, 0)),
            pl.BlockSpec((H, H), lambda i: (0, 0)),
            pl.BlockSpec((1, H), lambda i: (0, 0)),
            pl.BlockSpec((H, H), lambda i: (0, 0)),
            pl.BlockSpec((1, H), lambda i: (0, 0)),
            pl.BlockSpec((1, H), lambda i: (0, 0)),
            pl.BlockSpec((1, H), lambda i: (0, 0)),
        ],
        out_specs=pl.BlockSpec((blk, H), lambda i: (i, 0)),
        out_shape=jax.ShapeDtypeStruct((N, H), F32),
    )(x, p, q, u1a, u1b, b1, u2, b2, g, b)


def kernel(x, edge_index, edge_attr, coords,
           msg_W1, msg_b1, msg_W2, msg_b2,
           upd_W1, upd_b1, upd_W2, upd_b2,
           ln_g, ln_b):
    src = edge_index[0]
    dst = edge_index[1]
    ws = msg_W1[:H]
    wd = msg_W1[H:2 * H]
    we = msg_W1[2 * H:2 * H + EF]
    wc = msg_W1[2 * H + EF:]
    coords_p = jnp.pad(coords, ((0, 0), (0, 5)))
    wc_p = jnp.pad(wc, ((0, 5), (0, 0)))

    a, b = _prep(x, coords_p, ws, wd, wc_p, msg_b1.reshape(1, H))
    eh = E // 2
    b2r = msg_b2.reshape(1, H)
    ga1, gb1 = _gather2_h1(a, b, src, dst)
    msg1 = _edge(ga1, gb1, edge_attr, we, msg_W2, b2r, 0)
    ga2, gb2 = _gather2_h2(a, b, src, dst)
    msg2 = _edge(ga2, gb2, edge_attr, we, msg_W2, b2r, eh // 3200)
    p = _scatter_h1(msg1, dst).reshape(2, N, H)
    q = _scatter_h2(msg2, dst).reshape(2, N, H)
    return _node(x, p, q, upd_W1[:H], upd_W1[H:], upd_b1.reshape(1, H),
                 upd_W2, upd_b2.reshape(1, H),
                 ln_g.reshape(1, H), ln_b.reshape(1, H))


# speedup vs baseline: 1.7873x; 1.0003x over previous
import functools

import jax
import jax.numpy as jnp
from jax import lax
from jax.experimental import pallas as pl
from jax.experimental.pallas import tpu as pltpu
from jax.experimental.pallas import tpu_sc as plsc

N = 10000
E = 320000
H = 128
EF = 16

NC = 2
NS = 16
NW = NC * NS
EPW = E // NW
CH = 80
NCHUNK = EPW // CH
RCP = 80
NRC = N // RCP
F32 = jnp.float32


def _prep_body(x_ref, c_ref, ws_ref, wd_ref, wc_ref, b1_ref, a_ref, b_ref):
    x = x_ref[...]
    cw = jnp.dot(c_ref[...], wc_ref[...], preferred_element_type=F32)
    a_ref[...] = jnp.dot(x, ws_ref[...], preferred_element_type=F32) - cw
    b_ref[...] = jnp.dot(x, wd_ref[...], preferred_element_type=F32) + cw + b1_ref[...]


def _prep(x, coords_p, ws, wd, wc_p, b1):
    blk = 1000
    grid = N // blk
    return pl.pallas_call(
        _prep_body,
        grid=(grid,),
        in_specs=[
            pl.BlockSpec((blk, H), lambda i: (i, 0)),
            pl.BlockSpec((blk, 8), lambda i: (i, 0)),
            pl.BlockSpec((H, H), lambda i: (0, 0)),
            pl.BlockSpec((H, H), lambda i: (0, 0)),
            pl.BlockSpec((8, H), lambda i: (0, 0)),
            pl.BlockSpec((1, H), lambda i: (0, 0)),
        ],
        out_specs=[
            pl.BlockSpec((blk, H), lambda i: (i, 0)),
            pl.BlockSpec((blk, H), lambda i: (i, 0)),
        ],
        out_shape=[
            jax.ShapeDtypeStruct((N, H), F32),
            jax.ShapeDtypeStruct((N, H), F32),
        ],
    )(x, coords_p, ws, wd, wc_p, b1)


CHG = 40
NSLG = 5


def _make_gather2(ne, eoff):
    ept = ne // NS
    ncg = ept // CHG

    def _gather_body(a_hbm, b_hbm, src_hbm, dst_hbm, ga_hbm, gb_hbm, *s):
        idxv = s[0]
        rows = s[1:1 + NSLG]
        tbl = s[1 + NSLG]
        sg = s[2 + NSLG:2 + 2 * NSLG]
        sw = s[2 + 2 * NSLG:2 + 3 * NSLG]
        stl = s[2 + 3 * NSLG]
        cid = lax.axis_index("c")
        sid = lax.axis_index("s")
        base = sid * ept

        nmine = (NRC - 1 - sid) // NS + 1

        def tload(j, c2):
            r0 = pl.multiple_of((sid + j * NS) * RCP, 8)

            @pl.when(cid == 0)
            def _():
                pltpu.make_async_copy(a_hbm.at[pl.ds(r0, RCP)],
                                      tbl.at[pl.ds(r0, RCP)], stl).start()

            @pl.when(cid == 1)
            def _():
                pltpu.make_async_copy(b_hbm.at[pl.ds(r0, RCP)],
                                      tbl.at[pl.ds(r0, RCP)], stl).start()

            return c2

        lax.fori_loop(0, nmine, tload, 0)

        @pl.when(cid == 0)
        def _():
            pltpu.sync_copy(src_hbm.at[pl.ds(pl.multiple_of(eoff + base, 8), ept)],
                            idxv)

        @pl.when(cid == 1)
        def _():
            pltpu.sync_copy(dst_hbm.at[pl.ds(pl.multiple_of(eoff + base, 8), ept)],
                            idxv)

        def twait(j, c2):
            r0 = pl.multiple_of((sid + j * NS) * RCP, 8)
            pltpu.make_async_copy(a_hbm.at[pl.ds(r0, RCP)],
                                  tbl.at[pl.ds(r0, RCP)], stl).wait()
            return c2

        lax.fori_loop(0, nmine, twait, 0)
        plsc.subcore_barrier()

        def off_of(t):
            return pl.multiple_of(base + t * CHG, 8)

        def islice(t):
            return idxv.at[pl.ds(pl.multiple_of(t * CHG, 8), CHG)]

        for b in range(NSLG):
            pltpu.make_async_copy(tbl.at[islice(b)], rows[b], sg[b]).start()

        def outer(q, carry):
            for b in range(NSLG):
                t = NSLG * q + b
                pltpu.make_async_copy(tbl.at[islice(t)], rows[b], sg[b]).wait()
                r_b = rows[b]
                o = pl.ds(off_of(t), CHG)

                @pl.when(cid == 0)
                def _():
                    pltpu.make_async_copy(r_b, ga_hbm.at[o], sw[b]).start()

                @pl.when(cid == 1)
                def _():
                    pltpu.make_async_copy(r_b, gb_hbm.at[o], sw[b]).start()

            for b in range(NSLG):
                t = NSLG * q + b
                r_b = rows[b]
                o = pl.ds(off_of(t), CHG)

                @pl.when(t + NSLG < ncg)
                def _():
                    @pl.when(cid == 0)
                    def _():
                        pltpu.make_async_copy(r_b, ga_hbm.at[o], sw[b]).wait()

                    @pl.when(cid == 1)
                    def _():
                        pltpu.make_async_copy(r_b, gb_hbm.at[o], sw[b]).wait()

                    pltpu.make_async_copy(tbl.at[islice(t + NSLG)], rows[b],
                                          sg[b]).start()

            return carry

        lax.fori_loop(0, ncg // NSLG, outer, 0)
        for b in range(NSLG):
            t = ncg - NSLG + b
            r_b = rows[b]
            o = pl.ds(off_of(t), CHG)

            @pl.when(cid == 0)
            def _():
                pltpu.make_async_copy(r_b, ga_hbm.at[o], sw[b]).wait()

            @pl.when(cid == 1)
            def _():
                pltpu.make_async_copy(r_b, gb_hbm.at[o], sw[b]).wait()

    return functools.partial(
        pl.kernel,
        out_type=[jax.ShapeDtypeStruct((ne, H), F32),
                  jax.ShapeDtypeStruct((ne, H), F32)],
        mesh=plsc.VectorSubcoreMesh(core_axis_name="c", subcore_axis_name="s",
                                    num_cores=NC, num_subcores=NS),
        scratch_types=(
            [pltpu.VMEM((ept,), jnp.int32)]
            + [pltpu.VMEM((CHG, H), F32) for _ in range(NSLG)]
            + [pltpu.VMEM_SHARED((N, H), F32)]
            + [pltpu.SemaphoreType.DMA for _ in range(2 * NSLG + 1)]
        ),
    )(_gather_body)


_gather2_h1 = _make_gather2(E // 2, 0)
_gather2_h2 = _make_gather2(E // 2, E // 2)


def _edge_body(ga_ref, gb_ref, ea_ref, we_ref, w2_ref, b2_ref, m_ref):
    h1 = jnp.maximum(
        ga_ref[...] + gb_ref[...]
        + jnp.dot(ea_ref[...], we_ref[...], preferred_element_type=F32),
        0.0)
    m_ref[...] = jnp.maximum(
        jnp.dot(h1, w2_ref[...], preferred_element_type=F32) + b2_ref[...],
        0.0)


def _edge(ga, gb, ea, we, w2, b2, boff):
    ne = ga.shape[0]
    blk = 3200
    grid = ne // blk
    return pl.pallas_call(
        _edge_body,
        grid=(grid,),
        in_specs=[
            pl.BlockSpec((blk, H), lambda i: (i, 0)),
            pl.BlockSpec((blk, H), lambda i: (i, 0)),
            pl.BlockSpec((blk, EF), lambda i: (i + boff, 0)),
            pl.BlockSpec((EF, H), lambda i: (0, 0)),
            pl.BlockSpec((H, H), lambda i: (0, 0)),
            pl.BlockSpec((1, H), lambda i: (0, 0)),
        ],
        out_specs=pl.BlockSpec((blk, H), lambda i: (i, 0)),
        out_shape=jax.ShapeDtypeStruct((ne, H), F32),
    )(ga, gb, ea, we, w2, b2)


CHS = 40
NSL = 5


def _make_scatter(ne, eoff):
    epw = ne // NW
    ncs = epw // CHS

    def _scatter_body(msg_hbm, dst_hbm, out_hbm, *s):
        idxs = s[0:NSL]
        mv = s[NSL:2 * NSL]
        zv = s[2 * NSL]
        accum = s[2 * NSL + 1]
        sli = s[2 * NSL + 2:2 * NSL + 2 + NSL]
        slm = s[2 * NSL + 2 + NSL:2 * NSL + 2 + 2 * NSL]
        sad = s[2 * NSL + 2 + 2 * NSL:2 * NSL + 2 + 3 * NSL]
        szd = s[2 * NSL + 2 + 3 * NSL]
        cid = lax.axis_index("c")
        sid = lax.axis_index("s")
        base = cid * (ne // NC) + sid * epw

        def off_of(t):
            return pl.multiple_of(base + t * CHS, 8)

        def doff_of(t):
            return pl.multiple_of(eoff + base + t * CHS, 8)

        def zrow(r, c2):
            for k in range(8):
                zv[r, pl.ds(k * 16, 16)] = jnp.zeros((16,), F32)
            return c2

        lax.fori_loop(0, RCP, zrow, 0)

        nmine = (NRC - 1 - sid) // NS + 1

        def zcopy(j, c2):
            r0 = pl.multiple_of((sid + j * NS) * RCP, 8)
            pltpu.make_async_copy(zv, accum.at[pl.ds(r0, RCP)], szd).start()
            return c2

        lax.fori_loop(0, nmine, zcopy, 0)

        def zwait(j, c2):
            r0 = pl.multiple_of((sid + j * NS) * RCP, 8)
            pltpu.make_async_copy(zv, accum.at[pl.ds(r0, RCP)], szd).wait()
            return c2

        lax.fori_loop(0, nmine, zwait, 0)
        plsc.subcore_barrier()

        for b in range(NSL):
            off = off_of(b)
            pltpu.make_async_copy(dst_hbm.at[pl.ds(doff_of(b), CHS)], idxs[b], sli[b]).start()
            pltpu.make_async_copy(msg_hbm.at[pl.ds(off, CHS)], mv[b], slm[b]).start()

        def outer(g, carry):
            for b in range(NSL):
                t = NSL * g + b
                off = off_of(t)
                pltpu.make_async_copy(dst_hbm.at[pl.ds(doff_of(t), CHS)], idxs[b], sli[b]).wait()
                pltpu.make_async_copy(msg_hbm.at[pl.ds(off, CHS)], mv[b], slm[b]).wait()
                pltpu.async_copy(mv[b], accum.at[idxs[b]], sad[b], add=True)
            for b in range(NSL):
                t = NSL * g + b

                @pl.when(t + NSL < ncs)
                def _():
                    noff = off_of(t + NSL)
                    pltpu.make_async_copy(mv[b], accum.at[idxs[b]], sad[b]).wait()
                    pltpu.make_async_copy(dst_hbm.at[pl.ds(doff_of(t + NSL), CHS)], idxs[b], sli[b]).start()
                    pltpu.make_async_copy(msg_hbm.at[pl.ds(noff, CHS)], mv[b], slm[b]).start()
            return carry

        lax.fori_loop(0, ncs // NSL, outer, 0)
        for b in range(NSL):
            pltpu.make_async_copy(mv[b], accum.at[idxs[b]], sad[b]).wait()
        plsc.subcore_barrier()

        def ocopy(j, c2):
            r0 = pl.multiple_of((sid + j * NS) * RCP, 8)
            pltpu.make_async_copy(
                accum.at[pl.ds(r0, RCP)],
                out_hbm.at[pl.ds(pl.multiple_of(cid * N + r0, 8), RCP)], szd).start()
            return c2

        lax.fori_loop(0, nmine, ocopy, 0)

        def owait(j, c2):
            r0 = pl.multiple_of((sid + j * NS) * RCP, 8)
            pltpu.make_async_copy(
                accum.at[pl.ds(r0, RCP)],
                out_hbm.at[pl.ds(pl.multiple_of(cid * N + r0, 8), RCP)], szd).wait()
            return c2

        lax.fori_loop(0, nmine, owait, 0)

    return functools.partial(
        pl.kernel,
        out_type=jax.ShapeDtypeStruct((2 * N, H), F32),
        mesh=plsc.VectorSubcoreMesh(core_axis_name="c", subcore_axis_name="s",
                                    num_cores=NC, num_subcores=NS),
        scratch_types=(
            [pltpu.VMEM((CHS,), jnp.int32) for _ in range(NSL)]
            + [pltpu.VMEM((CHS, H), F32) for _ in range(NSL)]
            + [pltpu.VMEM((RCP, H), F32), pltpu.VMEM_SHARED((N, H), F32)]
            + [pltpu.SemaphoreType.DMA for _ in range(3 * NSL + 1)]
        ),
    )(_scatter_body)


_scatter_h1 = _make_scatter(E // 2, 0)
_scatter_h2 = _make_scatter(E // 2, E // 2)


def _node_body(x_ref, p_ref, q_ref, u1a_ref, u1b_ref, b1_ref, u2_ref, b2_ref,
               g_ref, bb_ref, o_ref):
    x = x_ref[...]
    aggr = (p_ref[0, :, :] + p_ref[1, :, :]) + (q_ref[0, :, :] + q_ref[1, :, :])
    h = jnp.maximum(
        jnp.dot(x, u1a_ref[...], preferred_element_type=F32)
        + jnp.dot(aggr, u1b_ref[...], preferred_element_type=F32)
        + b1_ref[...], 0.0)
    o = jnp.maximum(jnp.dot(h, u2_ref[...], preferred_element_type=F32)
                    + b2_ref[...], 0.0)
    y = x + o
    mu = jnp.mean(y, axis=1, keepdims=True)
    var = jnp.mean((y - mu) * (y - mu), axis=1, keepdims=True)
    o_ref[...] = (y - mu) * lax.rsqrt(var + 1e-5) * g_ref[...] + bb_ref[...]


def _node(x, p, q, u1a, u1b, b1, u2, b2, g, b):
    blk = 1000
    grid = N // blk
    return pl.pallas_call(
        _node_body,
        grid=(grid,),
        in_specs=[
            pl.BlockSpec((blk, H), lambda i: (i, 0)),
            pl.BlockSpec((2, blk, H), lambda i: (0, i, 0)),
            pl.BlockSpec((2, blk, H), lambda i: (0, i, 0)),
            pl.BlockSpec((H, H), lambda i: (0, 0)),
            pl.BlockSpec((H, H), lambda i: (0, 0)),
            pl.BlockSpec((1, H), lambda i: (0, 0)),
            pl.BlockSpec((H, H), lambda i: (0, 0)),
            pl.BlockSpec((1, H), lambda i: (0, 0)),
            pl.BlockSpec((1, H), lambda i: (0, 0)),
            pl.BlockSpec((1, H), lambda i: (0, 0)),
        ],
        out_specs=pl.BlockSpec((blk, H), lambda i: (i, 0)),
        out_shape=jax.ShapeDtypeStruct((N, H), F32),
    )(x, p, q, u1a, u1b, b1, u2, b2, g, b)


def kernel(x, edge_index, edge_attr, coords,
           msg_W1, msg_b1, msg_W2, msg_b2,
           upd_W1, upd_b1, upd_W2, upd_b2,
           ln_g, ln_b):
    src = edge_index[0]
    dst = edge_index[1]
    ws = msg_W1[:H]
    wd = msg_W1[H:2 * H]
    we = msg_W1[2 * H:2 * H + EF]
    wc = msg_W1[2 * H + EF:]
    coords_p = jnp.pad(coords, ((0, 0), (0, 5)))
    wc_p = jnp.pad(wc, ((0, 5), (0, 0)))

    a, b = _prep(x, coords_p, ws, wd, wc_p, msg_b1.reshape(1, H))
    eh = E // 2
    b2r = msg_b2.reshape(1, H)
    ga1, gb1 = _gather2_h1(a, b, src, dst)
    msg1 = _edge(ga1, gb1, edge_attr, we, msg_W2, b2r, 0)
    ga2, gb2 = _gather2_h2(a, b, src, dst)
    msg2 = _edge(ga2, gb2, edge_attr, we, msg_W2, b2r, eh // 3200)
    p = _scatter_h1(msg1, dst).reshape(2, N, H)
    q = _scatter_h2(msg2, dst).reshape(2, N, H)
    return _node(x, p, q, upd_W1[:H], upd_W1[H:], upd_b1.reshape(1, H),
                 upd_W2, upd_b2.reshape(1, H),
                 ln_g.reshape(1, H), ln_b.reshape(1, H))


# speedup vs baseline: 1.7949x; 1.0042x over previous
import functools

import jax
import jax.numpy as jnp
from jax import lax
from jax.experimental import pallas as pl
from jax.experimental.pallas import tpu as pltpu
from jax.experimental.pallas import tpu_sc as plsc

N = 10000
E = 320000
H = 128
EF = 16

NC = 2
NS = 16
NW = NC * NS
EPW = E // NW
CH = 80
NCHUNK = EPW // CH
RCP = 80
NRC = N // RCP
F32 = jnp.float32


def _prep_body(x_ref, c_ref, w1_ref, b1_ref, a_ref, b_ref):
    x = x_ref[...]
    w1 = w1_ref[...]
    cw = jnp.dot(c_ref[...], w1[2 * H + EF:], preferred_element_type=F32)
    a_ref[...] = jnp.dot(x, w1[:H], preferred_element_type=F32) - cw
    b_ref[...] = (jnp.dot(x, w1[H:2 * H], preferred_element_type=F32) + cw
                  + b1_ref[...][None, :])


def _prep(x, coords, w1, b1):
    blk = 1000
    grid = N // blk
    return pl.pallas_call(
        _prep_body,
        grid=(grid,),
        in_specs=[
            pl.BlockSpec((blk, H), lambda i: (i, 0)),
            pl.BlockSpec((blk, 3), lambda i: (i, 0)),
            pl.BlockSpec((2 * H + EF + 3, H), lambda i: (0, 0)),
            pl.BlockSpec((H,), lambda i: (0,)),
        ],
        out_specs=[
            pl.BlockSpec((blk, H), lambda i: (i, 0)),
            pl.BlockSpec((blk, H), lambda i: (i, 0)),
        ],
        out_shape=[
            jax.ShapeDtypeStruct((N, H), F32),
            jax.ShapeDtypeStruct((N, H), F32),
        ],
    )(x, coords, w1, b1)


CHG = 40
NSLG = 5


def _make_gather2(ne, eoff):
    ept = ne // NS
    ncg = ept // CHG

    def _gather_body(a_hbm, b_hbm, src_hbm, dst_hbm, ga_hbm, gb_hbm, *s):
        idxv = s[0]
        rows = s[1:1 + NSLG]
        tbl = s[1 + NSLG]
        sg = s[2 + NSLG:2 + 2 * NSLG]
        sw = s[2 + 2 * NSLG:2 + 3 * NSLG]
        stl = s[2 + 3 * NSLG]
        cid = lax.axis_index("c")
        sid = lax.axis_index("s")
        base = sid * ept

        nmine = (NRC - 1 - sid) // NS + 1

        def tload(j, c2):
            r0 = pl.multiple_of((sid + j * NS) * RCP, 8)

            @pl.when(cid == 0)
            def _():
                pltpu.make_async_copy(a_hbm.at[pl.ds(r0, RCP)],
                                      tbl.at[pl.ds(r0, RCP)], stl).start()

            @pl.when(cid == 1)
            def _():
                pltpu.make_async_copy(b_hbm.at[pl.ds(r0, RCP)],
                                      tbl.at[pl.ds(r0, RCP)], stl).start()

            return c2

        lax.fori_loop(0, nmine, tload, 0)

        @pl.when(cid == 0)
        def _():
            pltpu.sync_copy(src_hbm.at[pl.ds(pl.multiple_of(eoff + base, 8), ept)],
                            idxv)

        @pl.when(cid == 1)
        def _():
            pltpu.sync_copy(dst_hbm.at[pl.ds(pl.multiple_of(eoff + base, 8), ept)],
                            idxv)

        def twait(j, c2):
            r0 = pl.multiple_of((sid + j * NS) * RCP, 8)
            pltpu.make_async_copy(a_hbm.at[pl.ds(r0, RCP)],
                                  tbl.at[pl.ds(r0, RCP)], stl).wait()
            return c2

        lax.fori_loop(0, nmine, twait, 0)
        plsc.subcore_barrier()

        def off_of(t):
            return pl.multiple_of(base + t * CHG, 8)

        def islice(t):
            return idxv.at[pl.ds(pl.multiple_of(t * CHG, 8), CHG)]

        for b in range(NSLG):
            pltpu.make_async_copy(tbl.at[islice(b)], rows[b], sg[b]).start()

        def outer(q, carry):
            for b in range(NSLG):
                t = NSLG * q + b
                pltpu.make_async_copy(tbl.at[islice(t)], rows[b], sg[b]).wait()
                r_b = rows[b]
                o = pl.ds(off_of(t), CHG)

                @pl.when(cid == 0)
                def _():
                    pltpu.make_async_copy(r_b, ga_hbm.at[o], sw[b]).start()

                @pl.when(cid == 1)
                def _():
                    pltpu.make_async_copy(r_b, gb_hbm.at[o], sw[b]).start()

            for b in range(NSLG):
                t = NSLG * q + b
                r_b = rows[b]
                o = pl.ds(off_of(t), CHG)

                @pl.when(t + NSLG < ncg)
                def _():
                    @pl.when(cid == 0)
                    def _():
                        pltpu.make_async_copy(r_b, ga_hbm.at[o], sw[b]).wait()

                    @pl.when(cid == 1)
                    def _():
                        pltpu.make_async_copy(r_b, gb_hbm.at[o], sw[b]).wait()

                    pltpu.make_async_copy(tbl.at[islice(t + NSLG)], rows[b],
                                          sg[b]).start()

            return carry

        lax.fori_loop(0, ncg // NSLG, outer, 0)
        for b in range(NSLG):
            t = ncg - NSLG + b
            r_b = rows[b]
            o = pl.ds(off_of(t), CHG)

            @pl.when(cid == 0)
            def _():
                pltpu.make_async_copy(r_b, ga_hbm.at[o], sw[b]).wait()

            @pl.when(cid == 1)
            def _():
                pltpu.make_async_copy(r_b, gb_hbm.at[o], sw[b]).wait()

    return functools.partial(
        pl.kernel,
        out_type=[jax.ShapeDtypeStruct((ne, H), F32),
                  jax.ShapeDtypeStruct((ne, H), F32)],
        mesh=plsc.VectorSubcoreMesh(core_axis_name="c", subcore_axis_name="s",
                                    num_cores=NC, num_subcores=NS),
        scratch_types=(
            [pltpu.VMEM((ept,), jnp.int32)]
            + [pltpu.VMEM((CHG, H), F32) for _ in range(NSLG)]
            + [pltpu.VMEM_SHARED((N, H), F32)]
            + [pltpu.SemaphoreType.DMA for _ in range(2 * NSLG + 1)]
        ),
    )(_gather_body)


_gather2_h1 = _make_gather2(E // 2, 0)
_gather2_h2 = _make_gather2(E // 2, E // 2)


def _edge_body(ga_ref, gb_ref, ea_ref, w1_ref, w2_ref, b2_ref, m_ref):
    we = w1_ref[...][2 * H:2 * H + EF]
    h1 = jnp.maximum(
        ga_ref[...] + gb_ref[...]
        + jnp.dot(ea_ref[...], we, preferred_element_type=F32),
        0.0)
    m_ref[...] = jnp.maximum(
        jnp.dot(h1, w2_ref[...], preferred_element_type=F32)
        + b2_ref[...][None, :],
        0.0)


def _edge(ga, gb, ea, we, w2, b2, boff):
    ne = ga.shape[0]
    blk = 3200
    grid = ne // blk
    return pl.pallas_call(
        _edge_body,
        grid=(grid,),
        in_specs=[
            pl.BlockSpec((blk, H), lambda i: (i, 0)),
            pl.BlockSpec((blk, H), lambda i: (i, 0)),
            pl.BlockSpec((blk, EF), lambda i: (i + boff, 0)),
            pl.BlockSpec((2 * H + EF + 3, H), lambda i: (0, 0)),
            pl.BlockSpec((H, H), lambda i: (0, 0)),
            pl.BlockSpec((H,), lambda i: (0,)),
        ],
        out_specs=pl.BlockSpec((blk, H), lambda i: (i, 0)),
        out_shape=jax.ShapeDtypeStruct((ne, H), F32),
    )(ga, gb, ea, we, w2, b2)


CHS = 40
NSL = 5


def _make_scatter(ne, eoff):
    epw = ne // NW
    ncs = epw // CHS

    def _scatter_body(msg_hbm, dst_hbm, out_hbm, *s):
        idxs = s[0:NSL]
        mv = s[NSL:2 * NSL]
        zv = s[2 * NSL]
        accum = s[2 * NSL + 1]
        sli = s[2 * NSL + 2:2 * NSL + 2 + NSL]
        slm = s[2 * NSL + 2 + NSL:2 * NSL + 2 + 2 * NSL]
        sad = s[2 * NSL + 2 + 2 * NSL:2 * NSL + 2 + 3 * NSL]
        szd = s[2 * NSL + 2 + 3 * NSL]
        cid = lax.axis_index("c")
        sid = lax.axis_index("s")
        base = cid * (ne // NC) + sid * epw

        def off_of(t):
            return pl.multiple_of(base + t * CHS, 8)

        def doff_of(t):
            return pl.multiple_of(eoff + base + t * CHS, 8)

        def zrow(r, c2):
            for k in range(8):
                zv[r, pl.ds(k * 16, 16)] = jnp.zeros((16,), F32)
            return c2

        lax.fori_loop(0, RCP, zrow, 0)

        nmine = (NRC - 1 - sid) // NS + 1

        def zcopy(j, c2):
            r0 = pl.multiple_of((sid + j * NS) * RCP, 8)
            pltpu.make_async_copy(zv, accum.at[pl.ds(r0, RCP)], szd).start()
            return c2

        lax.fori_loop(0, nmine, zcopy, 0)

        def zwait(j, c2):
            r0 = pl.multiple_of((sid + j * NS) * RCP, 8)
            pltpu.make_async_copy(zv, accum.at[pl.ds(r0, RCP)], szd).wait()
            return c2

        lax.fori_loop(0, nmine, zwait, 0)
        plsc.subcore_barrier()

        for b in range(NSL):
            off = off_of(b)
            pltpu.make_async_copy(dst_hbm.at[pl.ds(doff_of(b), CHS)], idxs[b], sli[b]).start()
            pltpu.make_async_copy(msg_hbm.at[pl.ds(off, CHS)], mv[b], slm[b]).start()

        def outer(g, carry):
            for b in range(NSL):
                t = NSL * g + b
                off = off_of(t)
                pltpu.make_async_copy(dst_hbm.at[pl.ds(doff_of(t), CHS)], idxs[b], sli[b]).wait()
                pltpu.make_async_copy(msg_hbm.at[pl.ds(off, CHS)], mv[b], slm[b]).wait()
                pltpu.async_copy(mv[b], accum.at[idxs[b]], sad[b], add=True)
            for b in range(NSL):
                t = NSL * g + b

                @pl.when(t + NSL < ncs)
                def _():
                    noff = off_of(t + NSL)
                    pltpu.make_async_copy(mv[b], accum.at[idxs[b]], sad[b]).wait()
                    pltpu.make_async_copy(dst_hbm.at[pl.ds(doff_of(t + NSL), CHS)], idxs[b], sli[b]).start()
                    pltpu.make_async_copy(msg_hbm.at[pl.ds(noff, CHS)], mv[b], slm[b]).start()
            return carry

        lax.fori_loop(0, ncs // NSL, outer, 0)
        for b in range(NSL):
            pltpu.make_async_copy(mv[b], accum.at[idxs[b]], sad[b]).wait()
        plsc.subcore_barrier()

        def ocopy(j, c2):
            r0 = pl.multiple_of((sid + j * NS) * RCP, 8)
            pltpu.make_async_copy(
                accum.at[pl.ds(r0, RCP)],
                out_hbm.at[pl.ds(pl.multiple_of(cid * N + r0, 8), RCP)], szd).start()
            return c2

        lax.fori_loop(0, nmine, ocopy, 0)

        def owait(j, c2):
            r0 = pl.multiple_of((sid + j * NS) * RCP, 8)
            pltpu.make_async_copy(
                accum.at[pl.ds(r0, RCP)],
                out_hbm.at[pl.ds(pl.multiple_of(cid * N + r0, 8), RCP)], szd).wait()
            return c2

        lax.fori_loop(0, nmine, owait, 0)

    return functools.partial(
        pl.kernel,
        out_type=jax.ShapeDtypeStruct((2 * N, H), F32),
        mesh=plsc.VectorSubcoreMesh(core_axis_name="c", subcore_axis_name="s",
                                    num_cores=NC, num_subcores=NS),
        scratch_types=(
            [pltpu.VMEM((CHS,), jnp.int32) for _ in range(NSL)]
            + [pltpu.VMEM((CHS, H), F32) for _ in range(NSL)]
            + [pltpu.VMEM((RCP, H), F32), pltpu.VMEM_SHARED((N, H), F32)]
            + [pltpu.SemaphoreType.DMA for _ in range(3 * NSL + 1)]
        ),
    )(_scatter_body)


_scatter_h1 = _make_scatter(E // 2, 0)
_scatter_h2 = _make_scatter(E // 2, E // 2)


def _node_body(x_ref, p_ref, q_ref, u1_ref, b1_ref, u2_ref, b2_ref,
               g_ref, bb_ref, o_ref):
    x = x_ref[...]
    u1 = u1_ref[...]
    aggr = (p_ref[0, :, :] + p_ref[1, :, :]) + (q_ref[0, :, :] + q_ref[1, :, :])
    h = jnp.maximum(
        jnp.dot(x, u1[:H], preferred_element_type=F32)
        + jnp.dot(aggr, u1[H:], preferred_element_type=F32)
        + b1_ref[...][None, :], 0.0)
    o = jnp.maximum(jnp.dot(h, u2_ref[...], preferred_element_type=F32)
                    + b2_ref[...][None, :], 0.0)
    y = x + o
    mu = jnp.mean(y, axis=1, keepdims=True)
    var = jnp.mean((y - mu) * (y - mu), axis=1, keepdims=True)
    o_ref[...] = ((y - mu) * lax.rsqrt(var + 1e-5) * g_ref[...][None, :]
                  + bb_ref[...][None, :])


def _node(x, p, q, u1, b1, u2, b2, g, b):
    blk = 1000
    grid = N // blk
    return pl.pallas_call(
        _node_body,
        grid=(grid,),
        in_specs=[
            pl.BlockSpec((blk, H), lambda i: (i, 0)),
            pl.BlockSpec((2, blk, H), lambda i: (0, i, 0)),
            pl.BlockSpec((2, blk, H), lambda i: (0, i, 0)),
            pl.BlockSpec((2 * H, H), lambda i: (0, 0)),
            pl.BlockSpec((H,), lambda i: (0,)),
            pl.BlockSpec((H, H), lambda i: (0, 0)),
            pl.BlockSpec((H,), lambda i: (0,)),
            pl.BlockSpec((H,), lambda i: (0,)),
            pl.BlockSpec((H,), lambda i: (0,)),
        ],
        out_specs=pl.BlockSpec((blk, H), lambda i: (i, 0)),
        out_shape=jax.ShapeDtypeStruct((N, H), F32),
    )(x, p, q, u1, b1, u2, b2, g, b)


def kernel(x, edge_index, edge_attr, coords,
           msg_W1, msg_b1, msg_W2, msg_b2,
           upd_W1, upd_b1, upd_W2, upd_b2,
           ln_g, ln_b):
    src = edge_index[0]
    dst = edge_index[1]
    a, b = _prep(x, coords, msg_W1, msg_b1)
    eh = E // 2
    ga1, gb1 = _gather2_h1(a, b, src, dst)
    msg1 = _edge(ga1, gb1, edge_attr, msg_W1, msg_W2, msg_b2, 0)
    ga2, gb2 = _gather2_h2(a, b, src, dst)
    msg2 = _edge(ga2, gb2, edge_attr, msg_W1, msg_W2, msg_b2, eh // 3200)
    p = _scatter_h1(msg1, dst).reshape(2, N, H)
    q = _scatter_h2(msg2, dst).reshape(2, N, H)
    return _node(x, p, q, upd_W1, upd_b1, upd_W2, upd_b2, ln_g, ln_b)
